# Initial kernel scaffold; baseline (speedup 1.0000x reference)
import jax
import jax.numpy as jnp
from jax.experimental import pallas as pl


def kernel(edge_index, emb, W1, b1, W2, b2, gamma, beta):
    raise NotImplementedError("write your pallas kernel here")


# speedup vs baseline: 10.7211x; 10.7211x over previous
import functools

import jax
import jax.numpy as jnp
from jax import lax
from jax.experimental import pallas as pl
from jax.experimental.pallas import tpu as pltpu
from jax.experimental.pallas import tpu_sc as plsc

N = 10000
E = 160000
D = 256
H = 512
HALF = 128
NCORE = 2
NSUB = 16
RPS = 624
TAIL = 16
TAIL_OFF = NSUB * RPS
CHUNK = 128
NCHUNKS = E // CHUNK
RBLK = 1000
NB = N // RBLK

_MESH = dict(mesh=plsc.VectorSubcoreMesh(core_axis_name="c", subcore_axis_name="s"))


def _sc_degree(dst, zeros128, ones128):
    @functools.partial(
        pl.kernel,
        out_type=jax.ShapeDtypeStruct((NCORE, N, HALF), jnp.float32),
        scratch_types=[
            pltpu.VMEM_SHARED((N, HALF), jnp.float32),
            pltpu.VMEM((CHUNK,), jnp.int32),
            pltpu.VMEM((CHUNK, HALF), jnp.float32),
        ],
        **_MESH,
    )
    def deg_kernel(dst_ref, z_ref, ones_ref, out_ref, acc_sh, didx, ones_v):
        c = lax.axis_index("c")
        s = lax.axis_index("s")
        w = c * NSUB + s
        pltpu.sync_copy(ones_ref, ones_v)
        pltpu.sync_copy(z_ref, acc_sh.at[pl.ds(s * RPS, RPS)])

        @pl.when(s == NSUB - 1)
        def _():
            pltpu.sync_copy(z_ref.at[pl.ds(0, TAIL)],
                            acc_sh.at[pl.ds(TAIL_OFF, TAIL)])

        plsc.subcore_barrier()
        nk = (NCHUNKS - w + 2 * NSUB - 1) // (2 * NSUB)

        def body(k, carry):
            base = (w + k * 2 * NSUB) * CHUNK
            pltpu.sync_copy(dst_ref.at[pl.ds(base, CHUNK)], didx)
            pltpu.sync_copy(ones_v, acc_sh.at[didx], add=True)
            return carry

        lax.fori_loop(0, nk, body, 0)
        plsc.subcore_barrier()
        pltpu.sync_copy(acc_sh.at[pl.ds(s * RPS, RPS)],
                        out_ref.at[c, pl.ds(s * RPS, RPS)])

        @pl.when(s == NSUB - 1)
        def _():
            pltpu.sync_copy(acc_sh.at[pl.ds(TAIL_OFF, TAIL)],
                            out_ref.at[c, pl.ds(TAIL_OFF, TAIL)])

    return deg_kernel(dst, zeros128, ones128)


def _sc_propagate(t_flat, src, dst, zeros128):
    @functools.partial(
        pl.kernel,
        out_type=jax.ShapeDtypeStruct((NCORE * N, HALF), jnp.float32),
        scratch_types=[
            pltpu.VMEM_SHARED((N, HALF), jnp.float32),
            pltpu.VMEM((CHUNK,), jnp.int32),
            pltpu.VMEM((CHUNK,), jnp.int32),
            pltpu.VMEM((CHUNK, HALF), jnp.float32),
            pltpu.SemaphoreType.DMA,
        ],
        **_MESH,
    )
    def mp_kernel(t_ref, src_ref, dst_ref, z_ref, out_ref,
                  acc_sh, sidx, didx, rows, sem):
        c = lax.axis_index("c")
        s = lax.axis_index("s")
        pltpu.sync_copy(z_ref, acc_sh.at[pl.ds(s * RPS, RPS)])

        @pl.when(s == NSUB - 1)
        def _():
            pltpu.sync_copy(z_ref.at[pl.ds(0, TAIL)],
                            acc_sh.at[pl.ds(TAIL_OFF, TAIL)])

        plsc.subcore_barrier()
        c_off = c * N
        nk = (NCHUNKS - s + NSUB - 1) // NSUB

        def body(k, carry):
            base = (s + k * NSUB) * CHUNK
            pltpu.sync_copy(src_ref.at[pl.ds(base, CHUNK)], sidx)
            pltpu.sync_copy(dst_ref.at[pl.ds(base, CHUNK)], didx)
            for g in range(CHUNK // 16):
                sl = pl.ds(g * 16, 16)
                sidx[sl] = sidx[sl] + c_off
            pltpu.async_copy(t_ref.at[sidx], rows, sem).wait()
            pltpu.sync_copy(rows, acc_sh.at[didx], add=True)
            return carry

        lax.fori_loop(0, nk, body, 0)
        plsc.subcore_barrier()
        pltpu.sync_copy(acc_sh.at[pl.ds(s * RPS, RPS)],
                        out_ref.at[pl.ds(c * N + s * RPS, RPS)])

        @pl.when(s == NSUB - 1)
        def _():
            pltpu.sync_copy(acc_sh.at[pl.ds(TAIL_OFF, TAIL)],
                            out_ref.at[pl.ds(c * N + TAIL_OFF, TAIL)])

    return mp_kernel(t_flat, src, dst, zeros128)


def _tc_matmul(emb, W1, W2, b1p):
    def kern(emb_ref, w1_ref, w2_ref, b1_ref, mf_ref, crow_ref, w12):
        i = pl.program_id(1)

        @pl.when(i == 0)
        def _():
            w12[...] = jnp.dot(w1_ref[...], w2_ref[...],
                               preferred_element_type=jnp.float32)
            crow_ref[...] = jnp.dot(b1_ref[...], w2_ref[...],
                                    preferred_element_type=jnp.float32)

        mf_ref[...] = jnp.dot(emb_ref[...], w12[...],
                              preferred_element_type=jnp.float32)

    return pl.pallas_call(
        kern,
        grid=(NCORE, NB),
        in_specs=[
            pl.BlockSpec((RBLK, D), lambda c, i: (i, 0)),
            pl.BlockSpec((D, H), lambda c, i: (0, 0)),
            pl.BlockSpec((H, HALF), lambda c, i: (0, c)),
            pl.BlockSpec((8, H), lambda c, i: (0, 0)),
        ],
        out_specs=[
            pl.BlockSpec((RBLK, HALF), lambda c, i: (c * NB + i, 0)),
            pl.BlockSpec((8, HALF), lambda c, i: (0, c)),
        ],
        out_shape=[
            jax.ShapeDtypeStruct((NCORE * N, HALF), jnp.float32),
            jax.ShapeDtypeStruct((8, D), jnp.float32),
        ],
        scratch_shapes=[pltpu.VMEM((D, HALF), jnp.float32)],
    )(emb, W1, W2, b1p)


def _tc_scale(mf, degp):
    def kern(mf_ref, deg_ref, t_ref, dv_ref):
        d = deg_ref[0, :, 0:1] + deg_ref[1, :, 0:1] + 1.0
        d = jnp.maximum(d, 1e-12)
        dinv = lax.rsqrt(d)
        dinv2 = 1.0 / d
        t_ref[...] = mf_ref[...] * dinv
        dv_ref[...] = jnp.concatenate(
            [dinv, dinv2] + [dinv] * 14, axis=1)

    return pl.pallas_call(
        kern,
        grid=(NCORE, NB),
        in_specs=[
            pl.BlockSpec((RBLK, HALF), lambda c, i: (c * NB + i, 0)),
            pl.BlockSpec((NCORE, RBLK, HALF), lambda c, i: (0, i, 0)),
        ],
        out_specs=[
            pl.BlockSpec((RBLK, HALF), lambda c, i: (c * NB + i, 0)),
            pl.BlockSpec((RBLK, 16), lambda c, i: (i, 0)),
        ],
        out_shape=[
            jax.ShapeDtypeStruct((NCORE * N, HALF), jnp.float32),
            jax.ShapeDtypeStruct((N, 16), jnp.float32),
        ],
    )(mf, degp)


def _tc_combine(s1, t, dinvs, crow):
    def kern(s1_ref, t_ref, dv_ref, crow_ref, t2_ref):
        dinv = dv_ref[:, 0:1]
        dinv2 = dv_ref[:, 1:2]
        t2_ref[...] = (s1_ref[...] + t_ref[...]) * dinv2 + crow_ref[0:1, :] * dinv

    return pl.pallas_call(
        kern,
        grid=(NCORE, NB),
        in_specs=[
            pl.BlockSpec((RBLK, HALF), lambda c, i: (c * NB + i, 0)),
            pl.BlockSpec((RBLK, HALF), lambda c, i: (c * NB + i, 0)),
            pl.BlockSpec((RBLK, 16), lambda c, i: (i, 0)),
            pl.BlockSpec((8, HALF), lambda c, i: (0, c)),
        ],
        out_specs=pl.BlockSpec((RBLK, HALF), lambda c, i: (c * NB + i, 0)),
        out_shape=jax.ShapeDtypeStruct((NCORE * N, HALF), jnp.float32),
    )(s1, t, dinvs, crow)


def _tc_finalize(s2, t2, dinvs):
    def kern(s2_ref, t2_ref, dv_ref, y_ref, st_ref, acc):
        i = pl.program_id(1)
        dinv = dv_ref[:, 0:1]
        y = (s2_ref[...] + t2_ref[...]) * dinv
        y_ref[...] = y

        @pl.when(i == 0)
        def _():
            acc[...] = jnp.zeros_like(acc)

        acc[0:1, :] += jnp.sum(y, axis=0, keepdims=True)
        acc[1:2, :] += jnp.sum(y * y, axis=0, keepdims=True)

        @pl.when(i == NB - 1)
        def _():
            st_ref[0, :, :] = acc[...]

    return pl.pallas_call(
        kern,
        grid=(NCORE, NB),
        in_specs=[
            pl.BlockSpec((RBLK, HALF), lambda c, i: (c * NB + i, 0)),
            pl.BlockSpec((RBLK, HALF), lambda c, i: (c * NB + i, 0)),
            pl.BlockSpec((RBLK, 16), lambda c, i: (i, 0)),
        ],
        out_specs=[
            pl.BlockSpec((RBLK, HALF), lambda c, i: (i, c)),
            pl.BlockSpec((1, 8, HALF), lambda c, i: (c, 0, 0)),
        ],
        out_shape=[
            jax.ShapeDtypeStruct((N, D), jnp.float32),
            jax.ShapeDtypeStruct((NCORE, 8, HALF), jnp.float32),
        ],
        scratch_shapes=[pltpu.VMEM((8, HALF), jnp.float32)],
    )(s2, t2, dinvs)


def _tc_batchnorm(y2, stats, gamma, beta):
    def kern(y_ref, st_ref, g_ref, b_ref, out_ref):
        sums = jnp.concatenate([st_ref[0, 0:1, :], st_ref[1, 0:1, :]], axis=1)
        sqs = jnp.concatenate([st_ref[0, 1:2, :], st_ref[1, 1:2, :]], axis=1)
        mu = sums * (1.0 / N)
        var = sqs * (1.0 / N) - mu * mu
        scale = g_ref[...] * lax.rsqrt(var + 1e-5)
        out_ref[...] = y_ref[...] * scale + (b_ref[...] - mu * scale)

    return pl.pallas_call(
        kern,
        grid=(NB,),
        in_specs=[
            pl.BlockSpec((RBLK, D), lambda i: (i, 0)),
            pl.BlockSpec((NCORE, 8, HALF), lambda i: (0, 0, 0)),
            pl.BlockSpec((1, D), lambda i: (0, 0)),
            pl.BlockSpec((1, D), lambda i: (0, 0)),
        ],
        out_specs=pl.BlockSpec((RBLK, D), lambda i: (i, 0)),
        out_shape=jax.ShapeDtypeStruct((N, D), jnp.float32),
    )(y2, stats, gamma, beta)


def kernel(edge_index, emb, W1, b1, W2, b2, gamma, beta):
    del b2
    src = edge_index[0]
    dst = edge_index[1]
    zeros128 = jnp.zeros((RPS, HALF), jnp.float32)
    ones128 = jnp.ones((CHUNK, HALF), jnp.float32)
    b1p = jnp.broadcast_to(b1[None, :], (8, H))

    degp = _sc_degree(dst, zeros128, ones128)
    mf, crow = _tc_matmul(emb, W1, W2, b1p)
    t, dinvs = _tc_scale(mf, degp)
    s1 = _sc_propagate(t, src, dst, zeros128)
    t2 = _tc_combine(s1, t, dinvs, crow)
    s2 = _sc_propagate(t2, src, dst, zeros128)
    y2, stats = _tc_finalize(s2, t2, dinvs)
    return _tc_batchnorm(y2, stats, gamma[None, :], beta[None, :])


# speedup vs baseline: 14.8224x; 1.3825x over previous
import functools

import jax
import jax.numpy as jnp
from jax import lax
from jax.experimental import pallas as pl
from jax.experimental.pallas import tpu as pltpu
from jax.experimental.pallas import tpu_sc as plsc

N = 10000
E = 160000
D = 256
H = 512
HALF = 128
NCORE = 2
NSUB = 16
RPS = 624
TAIL = 16
TAIL_OFF = NSUB * RPS
CHUNK = 128
NCHUNKS = E // CHUNK
RBLK = 1000
NB = N // RBLK

_MESH = dict(mesh=plsc.VectorSubcoreMesh(core_axis_name="c", subcore_axis_name="s"))


def _sc_degree(dst, zeros128, ones128):
    @functools.partial(
        pl.kernel,
        out_type=jax.ShapeDtypeStruct((NCORE, N, HALF), jnp.float32),
        scratch_types=[
            pltpu.VMEM_SHARED((N, HALF), jnp.float32),
            pltpu.VMEM((CHUNK,), jnp.int32),
            pltpu.VMEM((CHUNK, HALF), jnp.float32),
        ],
        **_MESH,
    )
    def deg_kernel(dst_ref, z_ref, ones_ref, out_ref, acc_sh, didx, ones_v):
        c = lax.axis_index("c")
        s = lax.axis_index("s")
        w = c * NSUB + s
        pltpu.sync_copy(ones_ref, ones_v)
        pltpu.sync_copy(z_ref, acc_sh.at[pl.ds(s * RPS, RPS)])

        @pl.when(s == NSUB - 1)
        def _():
            pltpu.sync_copy(z_ref.at[pl.ds(0, TAIL)],
                            acc_sh.at[pl.ds(TAIL_OFF, TAIL)])

        plsc.subcore_barrier()
        nk = (NCHUNKS - w + 2 * NSUB - 1) // (2 * NSUB)

        def body(k, carry):
            base = (w + k * 2 * NSUB) * CHUNK
            pltpu.sync_copy(dst_ref.at[pl.ds(base, CHUNK)], didx)
            pltpu.sync_copy(ones_v, acc_sh.at[didx], add=True)
            return carry

        lax.fori_loop(0, nk, body, 0)
        plsc.subcore_barrier()
        pltpu.sync_copy(acc_sh.at[pl.ds(s * RPS, RPS)],
                        out_ref.at[c, pl.ds(s * RPS, RPS)])

        @pl.when(s == NSUB - 1)
        def _():
            pltpu.sync_copy(acc_sh.at[pl.ds(TAIL_OFF, TAIL)],
                            out_ref.at[c, pl.ds(TAIL_OFF, TAIL)])

    return deg_kernel(dst, zeros128, ones128)


def _sc_propagate(t_flat, src, dst, zeros128):
    @functools.partial(
        pl.kernel,
        out_type=jax.ShapeDtypeStruct((NCORE * N, HALF), jnp.float32),
        scratch_types=[
            pltpu.VMEM_SHARED((N, HALF), jnp.float32),
            pltpu.VMEM((CHUNK,), jnp.int32),
            pltpu.VMEM((CHUNK,), jnp.int32),
            pltpu.VMEM((CHUNK,), jnp.int32),
            pltpu.VMEM((CHUNK,), jnp.int32),
            pltpu.VMEM((CHUNK, HALF), jnp.float32),
            pltpu.VMEM((CHUNK, HALF), jnp.float32),
            pltpu.SemaphoreType.DMA,
            pltpu.SemaphoreType.DMA,
        ],
        **_MESH,
    )
    def mp_kernel(t_ref, src_ref, dst_ref, z_ref, out_ref,
                  acc_sh, sidx0, didx0, sidx1, didx1, rows0, rows1,
                  sem0, sem1):
        c = lax.axis_index("c")
        s = lax.axis_index("s")
        pltpu.sync_copy(z_ref, acc_sh.at[pl.ds(s * RPS, RPS)])

        @pl.when(s == NSUB - 1)
        def _():
            pltpu.sync_copy(z_ref.at[pl.ds(0, TAIL)],
                            acc_sh.at[pl.ds(TAIL_OFF, TAIL)])

        plsc.subcore_barrier()
        c_off = c * N
        nk = (NCHUNKS - s + NSUB - 1) // NSUB

        def load_idx(k, sidx, didx):
            base = (s + k * NSUB) * CHUNK
            pltpu.sync_copy(src_ref.at[pl.ds(base, CHUNK)], sidx)
            pltpu.sync_copy(dst_ref.at[pl.ds(base, CHUNK)], didx)
            for g in range(CHUNK // 16):
                sl = pl.ds(g * 16, 16)
                sidx[sl] = sidx[sl] + c_off

        load_idx(0, sidx0, didx0)
        cp0 = pltpu.async_copy(t_ref.at[sidx0], rows0, sem0)

        def body(k, carry):
            @pl.when(k % 2 == 0)
            def _():
                @pl.when(k + 1 < nk)
                def _():
                    load_idx(k + 1, sidx1, didx1)
                    pltpu.async_copy(t_ref.at[sidx1], rows1, sem1)
                pltpu.make_async_copy(t_ref.at[sidx0], rows0, sem0).wait()
                pltpu.sync_copy(rows0, acc_sh.at[didx0], add=True)

            @pl.when(k % 2 == 1)
            def _():
                @pl.when(k + 1 < nk)
                def _():
                    load_idx(k + 1, sidx0, didx0)
                    pltpu.async_copy(t_ref.at[sidx0], rows0, sem0)
                pltpu.make_async_copy(t_ref.at[sidx1], rows1, sem1).wait()
                pltpu.sync_copy(rows1, acc_sh.at[didx1], add=True)

            return carry

        lax.fori_loop(0, nk, body, 0)
        plsc.subcore_barrier()
        pltpu.sync_copy(acc_sh.at[pl.ds(s * RPS, RPS)],
                        out_ref.at[pl.ds(c * N + s * RPS, RPS)])

        @pl.when(s == NSUB - 1)
        def _():
            pltpu.sync_copy(acc_sh.at[pl.ds(TAIL_OFF, TAIL)],
                            out_ref.at[pl.ds(c * N + TAIL_OFF, TAIL)])

    return mp_kernel(t_flat, src, dst, zeros128)


def _tc_matmul(emb, W1, W2, b1p):
    def kern(emb_ref, w1_ref, w2_ref, b1_ref, mf_ref, crow_ref, w12):
        i = pl.program_id(1)

        @pl.when(i == 0)
        def _():
            w12[...] = jnp.dot(w1_ref[...], w2_ref[...],
                               preferred_element_type=jnp.float32)
            crow_ref[...] = jnp.dot(b1_ref[...], w2_ref[...],
                                    preferred_element_type=jnp.float32)

        mf_ref[...] = jnp.dot(emb_ref[...], w12[...],
                              preferred_element_type=jnp.float32)

    return pl.pallas_call(
        kern,
        grid=(NCORE, NB),
        in_specs=[
            pl.BlockSpec((RBLK, D), lambda c, i: (i, 0)),
            pl.BlockSpec((D, H), lambda c, i: (0, 0)),
            pl.BlockSpec((H, HALF), lambda c, i: (0, c)),
            pl.BlockSpec((8, H), lambda c, i: (0, 0)),
        ],
        out_specs=[
            pl.BlockSpec((RBLK, HALF), lambda c, i: (c * NB + i, 0)),
            pl.BlockSpec((8, HALF), lambda c, i: (0, c)),
        ],
        out_shape=[
            jax.ShapeDtypeStruct((NCORE * N, HALF), jnp.float32),
            jax.ShapeDtypeStruct((8, D), jnp.float32),
        ],
        scratch_shapes=[pltpu.VMEM((D, HALF), jnp.float32)],
    )(emb, W1, W2, b1p)


def _tc_scale(mf, degp):
    def kern(mf_ref, deg_ref, t_ref, dv_ref):
        d = deg_ref[0, :, 0:1] + deg_ref[1, :, 0:1] + 1.0
        d = jnp.maximum(d, 1e-12)
        dinv = lax.rsqrt(d)
        dinv2 = 1.0 / d
        t_ref[...] = mf_ref[...] * dinv
        dv_ref[...] = jnp.concatenate(
            [dinv, dinv2] + [dinv] * 14, axis=1)

    return pl.pallas_call(
        kern,
        grid=(NCORE, NB),
        in_specs=[
            pl.BlockSpec((RBLK, HALF), lambda c, i: (c * NB + i, 0)),
            pl.BlockSpec((NCORE, RBLK, HALF), lambda c, i: (0, i, 0)),
        ],
        out_specs=[
            pl.BlockSpec((RBLK, HALF), lambda c, i: (c * NB + i, 0)),
            pl.BlockSpec((RBLK, 16), lambda c, i: (i, 0)),
        ],
        out_shape=[
            jax.ShapeDtypeStruct((NCORE * N, HALF), jnp.float32),
            jax.ShapeDtypeStruct((N, 16), jnp.float32),
        ],
    )(mf, degp)


def _tc_combine(s1, t, dinvs, crow):
    def kern(s1_ref, t_ref, dv_ref, crow_ref, t2_ref):
        dinv = dv_ref[:, 0:1]
        dinv2 = dv_ref[:, 1:2]
        t2_ref[...] = (s1_ref[...] + t_ref[...]) * dinv2 + crow_ref[0:1, :] * dinv

    return pl.pallas_call(
        kern,
        grid=(NCORE, NB),
        in_specs=[
            pl.BlockSpec((RBLK, HALF), lambda c, i: (c * NB + i, 0)),
            pl.BlockSpec((RBLK, HALF), lambda c, i: (c * NB + i, 0)),
            pl.BlockSpec((RBLK, 16), lambda c, i: (i, 0)),
            pl.BlockSpec((8, HALF), lambda c, i: (0, c)),
        ],
        out_specs=pl.BlockSpec((RBLK, HALF), lambda c, i: (c * NB + i, 0)),
        out_shape=jax.ShapeDtypeStruct((NCORE * N, HALF), jnp.float32),
    )(s1, t, dinvs, crow)


def _tc_finalize(s2, t2, dinvs):
    def kern(s2_ref, t2_ref, dv_ref, y_ref, st_ref, acc):
        i = pl.program_id(1)
        dinv = dv_ref[:, 0:1]
        y = (s2_ref[...] + t2_ref[...]) * dinv
        y_ref[...] = y

        @pl.when(i == 0)
        def _():
            acc[...] = jnp.zeros_like(acc)

        acc[0:1, :] += jnp.sum(y, axis=0, keepdims=True)
        acc[1:2, :] += jnp.sum(y * y, axis=0, keepdims=True)

        @pl.when(i == NB - 1)
        def _():
            st_ref[0, :, :] = acc[...]

    return pl.pallas_call(
        kern,
        grid=(NCORE, NB),
        in_specs=[
            pl.BlockSpec((RBLK, HALF), lambda c, i: (c * NB + i, 0)),
            pl.BlockSpec((RBLK, HALF), lambda c, i: (c * NB + i, 0)),
            pl.BlockSpec((RBLK, 16), lambda c, i: (i, 0)),
        ],
        out_specs=[
            pl.BlockSpec((RBLK, HALF), lambda c, i: (i, c)),
            pl.BlockSpec((1, 8, HALF), lambda c, i: (c, 0, 0)),
        ],
        out_shape=[
            jax.ShapeDtypeStruct((N, D), jnp.float32),
            jax.ShapeDtypeStruct((NCORE, 8, HALF), jnp.float32),
        ],
        scratch_shapes=[pltpu.VMEM((8, HALF), jnp.float32)],
    )(s2, t2, dinvs)


def _tc_batchnorm(y2, stats, gamma, beta):
    def kern(y_ref, st_ref, g_ref, b_ref, out_ref):
        sums = jnp.concatenate([st_ref[0, 0:1, :], st_ref[1, 0:1, :]], axis=1)
        sqs = jnp.concatenate([st_ref[0, 1:2, :], st_ref[1, 1:2, :]], axis=1)
        mu = sums * (1.0 / N)
        var = sqs * (1.0 / N) - mu * mu
        scale = g_ref[...] * lax.rsqrt(var + 1e-5)
        out_ref[...] = y_ref[...] * scale + (b_ref[...] - mu * scale)

    return pl.pallas_call(
        kern,
        grid=(NB,),
        in_specs=[
            pl.BlockSpec((RBLK, D), lambda i: (i, 0)),
            pl.BlockSpec((NCORE, 8, HALF), lambda i: (0, 0, 0)),
            pl.BlockSpec((1, D), lambda i: (0, 0)),
            pl.BlockSpec((1, D), lambda i: (0, 0)),
        ],
        out_specs=pl.BlockSpec((RBLK, D), lambda i: (i, 0)),
        out_shape=jax.ShapeDtypeStruct((N, D), jnp.float32),
    )(y2, stats, gamma, beta)


def kernel(edge_index, emb, W1, b1, W2, b2, gamma, beta):
    del b2
    src = edge_index[0]
    dst = edge_index[1]
    zeros128 = jnp.zeros((RPS, HALF), jnp.float32)
    ones128 = jnp.ones((CHUNK, HALF), jnp.float32)
    b1p = jnp.broadcast_to(b1[None, :], (8, H))

    degp = _sc_degree(dst, zeros128, ones128)
    mf, crow = _tc_matmul(emb, W1, W2, b1p)
    t, dinvs = _tc_scale(mf, degp)
    s1 = _sc_propagate(t, src, dst, zeros128)
    t2 = _tc_combine(s1, t, dinvs, crow)
    s2 = _sc_propagate(t2, src, dst, zeros128)
    y2, stats = _tc_finalize(s2, t2, dinvs)
    return _tc_batchnorm(y2, stats, gamma[None, :], beta[None, :])


# speedup vs baseline: 15.7937x; 1.0655x over previous
import functools

import jax
import jax.numpy as jnp
from jax import lax
from jax.experimental import pallas as pl
from jax.experimental.pallas import tpu as pltpu
from jax.experimental.pallas import tpu_sc as plsc

N = 10000
E = 160000
D = 256
H = 512
HALF = 128
NCORE = 2
NSUB = 16
RPS = 624
TAIL = 16
TAIL_OFF = NSUB * RPS
CHUNK = 128
NCHUNKS = E // CHUNK
RBLK = 1000
NB = N // RBLK

_MESH = dict(mesh=plsc.VectorSubcoreMesh(core_axis_name="c", subcore_axis_name="s"))


def _sc_degree(dst, zeros128, ones128):
    @functools.partial(
        pl.kernel,
        out_type=jax.ShapeDtypeStruct((NCORE, N, HALF), jnp.float32),
        scratch_types=[
            pltpu.VMEM_SHARED((N, HALF), jnp.float32),
            pltpu.VMEM((CHUNK,), jnp.int32),
            pltpu.VMEM((CHUNK, HALF), jnp.float32),
        ],
        **_MESH,
    )
    def deg_kernel(dst_ref, z_ref, ones_ref, out_ref, acc_sh, didx, ones_v):
        c = lax.axis_index("c")
        s = lax.axis_index("s")
        w = c * NSUB + s
        pltpu.sync_copy(ones_ref, ones_v)
        pltpu.sync_copy(z_ref, acc_sh.at[pl.ds(s * RPS, RPS)])

        @pl.when(s == NSUB - 1)
        def _():
            pltpu.sync_copy(z_ref.at[pl.ds(0, TAIL)],
                            acc_sh.at[pl.ds(TAIL_OFF, TAIL)])

        plsc.subcore_barrier()
        nk = (NCHUNKS - w + 2 * NSUB - 1) // (2 * NSUB)

        def body(k, carry):
            base = (w + k * 2 * NSUB) * CHUNK
            pltpu.sync_copy(dst_ref.at[pl.ds(base, CHUNK)], didx)
            pltpu.sync_copy(ones_v, acc_sh.at[didx], add=True)
            return carry

        lax.fori_loop(0, nk, body, 0)
        plsc.subcore_barrier()
        pltpu.sync_copy(acc_sh.at[pl.ds(s * RPS, RPS)],
                        out_ref.at[c, pl.ds(s * RPS, RPS)])

        @pl.when(s == NSUB - 1)
        def _():
            pltpu.sync_copy(acc_sh.at[pl.ds(TAIL_OFF, TAIL)],
                            out_ref.at[c, pl.ds(TAIL_OFF, TAIL)])

    return deg_kernel(dst, zeros128, ones128)


PCHUNK = 64
NKC = 156
NKMAX = NKC + 4


def _sc_propagate(t_flat, src, dst, zeros128):
    @functools.partial(
        pl.kernel,
        out_type=jax.ShapeDtypeStruct((NCORE * N, HALF), jnp.float32),
        scratch_types=[
            pltpu.VMEM_SHARED((N, HALF), jnp.float32),
            pltpu.VMEM((NKMAX * PCHUNK,), jnp.int32),
            pltpu.VMEM((NKMAX * PCHUNK,), jnp.int32),
            pltpu.VMEM((PCHUNK,), jnp.int32),
            pltpu.VMEM((PCHUNK,), jnp.int32),
            pltpu.VMEM((PCHUNK,), jnp.int32),
            pltpu.VMEM((PCHUNK,), jnp.int32),
            pltpu.VMEM((PCHUNK, HALF), jnp.float32),
            pltpu.VMEM((PCHUNK, HALF), jnp.float32),
            pltpu.SemaphoreType.DMA,
            pltpu.SemaphoreType.DMA,
        ],
        **_MESH,
    )
    def mp_kernel(t_ref, src_ref, dst_ref, z_ref, out_ref,
                  acc_sh, sall, dall, sidx0, didx0, sidx1, didx1,
                  rows0, rows1, sem0, sem1):
        c = lax.axis_index("c")
        s = lax.axis_index("s")
        base_e = s * (NKC * PCHUNK)
        pltpu.sync_copy(src_ref.at[pl.ds(base_e, NKC * PCHUNK)],
                        sall.at[pl.ds(0, NKC * PCHUNK)])
        pltpu.sync_copy(dst_ref.at[pl.ds(base_e, NKC * PCHUNK)],
                        dall.at[pl.ds(0, NKC * PCHUNK)])

        @pl.when(s == NSUB - 1)
        def _():
            pltpu.sync_copy(src_ref.at[pl.ds(NSUB * NKC * PCHUNK, 4 * PCHUNK)],
                            sall.at[pl.ds(NKC * PCHUNK, 4 * PCHUNK)])
            pltpu.sync_copy(dst_ref.at[pl.ds(NSUB * NKC * PCHUNK, 4 * PCHUNK)],
                            dall.at[pl.ds(NKC * PCHUNK, 4 * PCHUNK)])

        pltpu.sync_copy(z_ref, acc_sh.at[pl.ds(s * RPS, RPS)])

        @pl.when(s == NSUB - 1)
        def _():
            pltpu.sync_copy(z_ref.at[pl.ds(0, TAIL)],
                            acc_sh.at[pl.ds(TAIL_OFF, TAIL)])

        plsc.subcore_barrier()
        c_off = c * N
        nk = jnp.where(s == NSUB - 1, NKMAX, NKC)

        def stage_idx(k, sidx, didx):
            for g in range(PCHUNK // 16):
                sl = pl.ds(k * PCHUNK + g * 16, 16)
                sidx[pl.ds(g * 16, 16)] = sall[sl] + c_off
                didx[pl.ds(g * 16, 16)] = dall[sl]

        stage_idx(0, sidx0, didx0)
        pltpu.async_copy(t_ref.at[sidx0], rows0, sem0)

        def body(k, carry):
            @pl.when(k % 2 == 0)
            def _():
                @pl.when(k + 1 < nk)
                def _():
                    stage_idx(k + 1, sidx1, didx1)
                    pltpu.async_copy(t_ref.at[sidx1], rows1, sem1)
                pltpu.make_async_copy(t_ref.at[sidx0], rows0, sem0).wait()
                pltpu.sync_copy(rows0, acc_sh.at[didx0], add=True)

            @pl.when(k % 2 == 1)
            def _():
                @pl.when(k + 1 < nk)
                def _():
                    stage_idx(k + 1, sidx0, didx0)
                    pltpu.async_copy(t_ref.at[sidx0], rows0, sem0)
                pltpu.make_async_copy(t_ref.at[sidx1], rows1, sem1).wait()
                pltpu.sync_copy(rows1, acc_sh.at[didx1], add=True)

            return carry

        lax.fori_loop(0, nk, body, 0)
        plsc.subcore_barrier()
        pltpu.sync_copy(acc_sh.at[pl.ds(s * RPS, RPS)],
                        out_ref.at[pl.ds(c * N + s * RPS, RPS)])

        @pl.when(s == NSUB - 1)
        def _():
            pltpu.sync_copy(acc_sh.at[pl.ds(TAIL_OFF, TAIL)],
                            out_ref.at[pl.ds(c * N + TAIL_OFF, TAIL)])

    return mp_kernel(t_flat, src, dst, zeros128)


def _tc_matmul(emb, W1, W2, b1p):
    def kern(emb_ref, w1_ref, w2_ref, b1_ref, mf_ref, crow_ref, w12):
        i = pl.program_id(1)

        @pl.when(i == 0)
        def _():
            w12[...] = jnp.dot(w1_ref[...], w2_ref[...],
                               preferred_element_type=jnp.float32)
            crow_ref[...] = jnp.dot(b1_ref[...], w2_ref[...],
                                    preferred_element_type=jnp.float32)

        mf_ref[...] = jnp.dot(emb_ref[...], w12[...],
                              preferred_element_type=jnp.float32)

    return pl.pallas_call(
        kern,
        grid=(NCORE, NB),
        in_specs=[
            pl.BlockSpec((RBLK, D), lambda c, i: (i, 0)),
            pl.BlockSpec((D, H), lambda c, i: (0, 0)),
            pl.BlockSpec((H, HALF), lambda c, i: (0, c)),
            pl.BlockSpec((8, H), lambda c, i: (0, 0)),
        ],
        out_specs=[
            pl.BlockSpec((RBLK, HALF), lambda c, i: (c * NB + i, 0)),
            pl.BlockSpec((8, HALF), lambda c, i: (0, c)),
        ],
        out_shape=[
            jax.ShapeDtypeStruct((NCORE * N, HALF), jnp.float32),
            jax.ShapeDtypeStruct((8, D), jnp.float32),
        ],
        scratch_shapes=[pltpu.VMEM((D, HALF), jnp.float32)],
    )(emb, W1, W2, b1p)


def _tc_scale(mf, degp):
    def kern(mf_ref, deg_ref, t_ref, dv_ref):
        d = deg_ref[0, :, 0:1] + deg_ref[1, :, 0:1] + 1.0
        d = jnp.maximum(d, 1e-12)
        dinv = lax.rsqrt(d)
        dinv2 = 1.0 / d
        t_ref[...] = mf_ref[...] * dinv
        dv_ref[...] = jnp.concatenate(
            [dinv, dinv2] + [dinv] * 14, axis=1)

    return pl.pallas_call(
        kern,
        grid=(NCORE, NB),
        in_specs=[
            pl.BlockSpec((RBLK, HALF), lambda c, i: (c * NB + i, 0)),
            pl.BlockSpec((NCORE, RBLK, HALF), lambda c, i: (0, i, 0)),
        ],
        out_specs=[
            pl.BlockSpec((RBLK, HALF), lambda c, i: (c * NB + i, 0)),
            pl.BlockSpec((RBLK, 16), lambda c, i: (i, 0)),
        ],
        out_shape=[
            jax.ShapeDtypeStruct((NCORE * N, HALF), jnp.float32),
            jax.ShapeDtypeStruct((N, 16), jnp.float32),
        ],
    )(mf, degp)


def _tc_combine(s1, t, dinvs, crow):
    def kern(s1_ref, t_ref, dv_ref, crow_ref, t2_ref):
        dinv = dv_ref[:, 0:1]
        dinv2 = dv_ref[:, 1:2]
        t2_ref[...] = (s1_ref[...] + t_ref[...]) * dinv2 + crow_ref[0:1, :] * dinv

    return pl.pallas_call(
        kern,
        grid=(NCORE, NB),
        in_specs=[
            pl.BlockSpec((RBLK, HALF), lambda c, i: (c * NB + i, 0)),
            pl.BlockSpec((RBLK, HALF), lambda c, i: (c * NB + i, 0)),
            pl.BlockSpec((RBLK, 16), lambda c, i: (i, 0)),
            pl.BlockSpec((8, HALF), lambda c, i: (0, c)),
        ],
        out_specs=pl.BlockSpec((RBLK, HALF), lambda c, i: (c * NB + i, 0)),
        out_shape=jax.ShapeDtypeStruct((NCORE * N, HALF), jnp.float32),
    )(s1, t, dinvs, crow)


def _tc_finalize(s2, t2, dinvs):
    def kern(s2_ref, t2_ref, dv_ref, y_ref, st_ref, acc):
        i = pl.program_id(1)
        dinv = dv_ref[:, 0:1]
        y = (s2_ref[...] + t2_ref[...]) * dinv
        y_ref[...] = y

        @pl.when(i == 0)
        def _():
            acc[...] = jnp.zeros_like(acc)

        acc[0:1, :] += jnp.sum(y, axis=0, keepdims=True)
        acc[1:2, :] += jnp.sum(y * y, axis=0, keepdims=True)

        @pl.when(i == NB - 1)
        def _():
            st_ref[0, :, :] = acc[...]

    return pl.pallas_call(
        kern,
        grid=(NCORE, NB),
        in_specs=[
            pl.BlockSpec((RBLK, HALF), lambda c, i: (c * NB + i, 0)),
            pl.BlockSpec((RBLK, HALF), lambda c, i: (c * NB + i, 0)),
            pl.BlockSpec((RBLK, 16), lambda c, i: (i, 0)),
        ],
        out_specs=[
            pl.BlockSpec((RBLK, HALF), lambda c, i: (i, c)),
            pl.BlockSpec((1, 8, HALF), lambda c, i: (c, 0, 0)),
        ],
        out_shape=[
            jax.ShapeDtypeStruct((N, D), jnp.float32),
            jax.ShapeDtypeStruct((NCORE, 8, HALF), jnp.float32),
        ],
        scratch_shapes=[pltpu.VMEM((8, HALF), jnp.float32)],
    )(s2, t2, dinvs)


def _tc_batchnorm(y2, stats, gamma, beta):
    def kern(y_ref, st_ref, g_ref, b_ref, out_ref):
        sums = jnp.concatenate([st_ref[0, 0:1, :], st_ref[1, 0:1, :]], axis=1)
        sqs = jnp.concatenate([st_ref[0, 1:2, :], st_ref[1, 1:2, :]], axis=1)
        mu = sums * (1.0 / N)
        var = sqs * (1.0 / N) - mu * mu
        scale = g_ref[...] * lax.rsqrt(var + 1e-5)
        out_ref[...] = y_ref[...] * scale + (b_ref[...] - mu * scale)

    return pl.pallas_call(
        kern,
        grid=(NB,),
        in_specs=[
            pl.BlockSpec((RBLK, D), lambda i: (i, 0)),
            pl.BlockSpec((NCORE, 8, HALF), lambda i: (0, 0, 0)),
            pl.BlockSpec((1, D), lambda i: (0, 0)),
            pl.BlockSpec((1, D), lambda i: (0, 0)),
        ],
        out_specs=pl.BlockSpec((RBLK, D), lambda i: (i, 0)),
        out_shape=jax.ShapeDtypeStruct((N, D), jnp.float32),
    )(y2, stats, gamma, beta)


def kernel(edge_index, emb, W1, b1, W2, b2, gamma, beta):
    del b2
    src = edge_index[0]
    dst = edge_index[1]
    zeros128 = jnp.zeros((RPS, HALF), jnp.float32)
    ones128 = jnp.ones((CHUNK, HALF), jnp.float32)
    b1p = jnp.broadcast_to(b1[None, :], (8, H))

    degp = _sc_degree(dst, zeros128, ones128)
    mf, crow = _tc_matmul(emb, W1, W2, b1p)
    t, dinvs = _tc_scale(mf, degp)
    s1 = _sc_propagate(t, src, dst, zeros128)
    t2 = _tc_combine(s1, t, dinvs, crow)
    s2 = _sc_propagate(t2, src, dst, zeros128)
    y2, stats = _tc_finalize(s2, t2, dinvs)
    return _tc_batchnorm(y2, stats, gamma[None, :], beta[None, :])


# speedup vs baseline: 19.1557x; 1.2129x over previous
import functools

import jax
import jax.numpy as jnp
from jax import lax
from jax.experimental import pallas as pl
from jax.experimental.pallas import tpu as pltpu
from jax.experimental.pallas import tpu_sc as plsc

N = 10000
E = 160000
D = 256
H = 512
HALF = 128
NCORE = 2
NSUB = 16
RPS = 624
TAIL = 16
TAIL_OFF = NSUB * RPS
CHUNK = 128
NCHUNKS = E // CHUNK
RBLK = 1000
NB = N // RBLK

_MESH = dict(mesh=plsc.VectorSubcoreMesh(core_axis_name="c", subcore_axis_name="s"))


DKC = 39
DKMAX = DKC + 1


def _sc_degree(dst, zeros128, ones128):
    @functools.partial(
        pl.kernel,
        out_type=jax.ShapeDtypeStruct((NCORE, N, HALF), jnp.float32),
        scratch_types=[
            pltpu.VMEM_SHARED((N, HALF), jnp.float32),
            pltpu.VMEM((DKMAX * CHUNK,), jnp.int32),
            pltpu.VMEM((CHUNK,), jnp.int32),
            pltpu.VMEM((CHUNK,), jnp.int32),
            pltpu.VMEM((CHUNK, HALF), jnp.float32),
            pltpu.SemaphoreType.DMA,
            pltpu.SemaphoreType.DMA,
        ],
        **_MESH,
    )
    def deg_kernel(dst_ref, z_ref, ones_ref, out_ref,
                   acc_sh, dall, di0, di1, ones_v, sem0, sem1):
        c = lax.axis_index("c")
        s = lax.axis_index("s")
        w = c * NSUB + s
        base_e = w * (DKC * CHUNK)
        pltpu.sync_copy(dst_ref.at[pl.ds(base_e, DKC * CHUNK)],
                        dall.at[pl.ds(0, DKC * CHUNK)])

        @pl.when(w < 2)
        def _():
            pltpu.sync_copy(
                dst_ref.at[pl.ds(32 * DKC * CHUNK + w * CHUNK, CHUNK)],
                dall.at[pl.ds(DKC * CHUNK, CHUNK)])

        pltpu.sync_copy(ones_ref, ones_v)
        pltpu.sync_copy(z_ref, acc_sh.at[pl.ds(s * RPS, RPS)])

        @pl.when(s == NSUB - 1)
        def _():
            pltpu.sync_copy(z_ref.at[pl.ds(0, TAIL)],
                            acc_sh.at[pl.ds(TAIL_OFF, TAIL)])

        plsc.subcore_barrier()
        nk = jnp.where(w < 2, DKMAX, DKC)

        def body(k, carry):
            for g in range(CHUNK // 16):
                di0[pl.ds(g * 16, 16)] = dall[pl.ds(k * CHUNK + g * 16, 16)]
            pltpu.sync_copy(ones_v, acc_sh.at[di0], add=True)
            return carry

        lax.fori_loop(0, nk, body, 0)
        plsc.subcore_barrier()
        pltpu.sync_copy(acc_sh.at[pl.ds(s * RPS, RPS)],
                        out_ref.at[c, pl.ds(s * RPS, RPS)])

        @pl.when(s == NSUB - 1)
        def _():
            pltpu.sync_copy(acc_sh.at[pl.ds(TAIL_OFF, TAIL)],
                            out_ref.at[c, pl.ds(TAIL_OFF, TAIL)])

    return deg_kernel(dst, zeros128, ones128)


PCHUNK = 64
NKC = 156
NKMAX = NKC + 4


def _sc_propagate(t_flat, src, dst, zeros128):
    @functools.partial(
        pl.kernel,
        out_type=jax.ShapeDtypeStruct((NCORE * N, HALF), jnp.float32),
        scratch_types=[
            pltpu.VMEM_SHARED((N, HALF), jnp.float32),
            pltpu.VMEM((NKMAX * PCHUNK,), jnp.int32),
            pltpu.VMEM((NKMAX * PCHUNK,), jnp.int32),
            pltpu.VMEM((PCHUNK,), jnp.int32),
            pltpu.VMEM((PCHUNK,), jnp.int32),
            pltpu.VMEM((PCHUNK,), jnp.int32),
            pltpu.VMEM((PCHUNK,), jnp.int32),
            pltpu.VMEM((PCHUNK,), jnp.int32),
            pltpu.VMEM((PCHUNK,), jnp.int32),
            pltpu.VMEM((PCHUNK, HALF), jnp.float32),
            pltpu.VMEM((PCHUNK, HALF), jnp.float32),
            pltpu.VMEM((PCHUNK, HALF), jnp.float32),
            pltpu.SemaphoreType.DMA,
            pltpu.SemaphoreType.DMA,
            pltpu.SemaphoreType.DMA,
        ],
        **_MESH,
    )
    def mp_kernel(t_ref, src_ref, dst_ref, z_ref, out_ref,
                  acc_sh, sall, dall, si0, di0, si1, di1, si2, di2,
                  r0, r1, r2, sem0, sem1, sem2):
        c = lax.axis_index("c")
        s = lax.axis_index("s")
        base_e = s * (NKC * PCHUNK)
        pltpu.sync_copy(src_ref.at[pl.ds(base_e, NKC * PCHUNK)],
                        sall.at[pl.ds(0, NKC * PCHUNK)])
        pltpu.sync_copy(dst_ref.at[pl.ds(base_e, NKC * PCHUNK)],
                        dall.at[pl.ds(0, NKC * PCHUNK)])

        @pl.when(s == NSUB - 1)
        def _():
            pltpu.sync_copy(src_ref.at[pl.ds(NSUB * NKC * PCHUNK, 4 * PCHUNK)],
                            sall.at[pl.ds(NKC * PCHUNK, 4 * PCHUNK)])
            pltpu.sync_copy(dst_ref.at[pl.ds(NSUB * NKC * PCHUNK, 4 * PCHUNK)],
                            dall.at[pl.ds(NKC * PCHUNK, 4 * PCHUNK)])

        pltpu.sync_copy(z_ref, acc_sh.at[pl.ds(s * RPS, RPS)])

        @pl.when(s == NSUB - 1)
        def _():
            pltpu.sync_copy(z_ref.at[pl.ds(0, TAIL)],
                            acc_sh.at[pl.ds(TAIL_OFF, TAIL)])

        plsc.subcore_barrier()
        c_off = c * N
        nk = jnp.where(s == NSUB - 1, NKMAX, NKC)
        bufs = ((si0, di0, r0, sem0), (si1, di1, r1, sem1), (si2, di2, r2, sem2))

        def stage_idx(k, sidx, didx):
            for g in range(PCHUNK // 16):
                sl = pl.ds(k * PCHUNK + g * 16, 16)
                sidx[pl.ds(g * 16, 16)] = sall[sl] + c_off
                didx[pl.ds(g * 16, 16)] = dall[sl]

        def start(k, b):
            sidx, didx, rows, sem = bufs[b]
            stage_idx(k, sidx, didx)
            pltpu.async_copy(t_ref.at[sidx], rows, sem)

        start(0, 0)
        start(1, 1)

        def body(k, carry):
            for b in range(3):
                @pl.when(k % 3 == b)
                def _():
                    sidx, didx, rows, sem = bufs[b]

                    @pl.when(k + 2 < nk)
                    def _():
                        start(k + 2, (b + 2) % 3)

                    pltpu.make_async_copy(t_ref.at[sidx], rows, sem).wait()
                    pltpu.sync_copy(rows, acc_sh.at[didx], add=True)

            return carry

        lax.fori_loop(0, nk, body, 0)
        plsc.subcore_barrier()
        pltpu.sync_copy(acc_sh.at[pl.ds(s * RPS, RPS)],
                        out_ref.at[pl.ds(c * N + s * RPS, RPS)])

        @pl.when(s == NSUB - 1)
        def _():
            pltpu.sync_copy(acc_sh.at[pl.ds(TAIL_OFF, TAIL)],
                            out_ref.at[pl.ds(c * N + TAIL_OFF, TAIL)])

    return mp_kernel(t_flat, src, dst, zeros128)


def _tc_matmul_scale(emb, W1, W2, b1p, degp):
    def kern(emb_ref, w1_ref, w2_ref, b1_ref, deg_ref,
             t_ref, crow_ref, dv_ref, w12):
        i = pl.program_id(1)

        @pl.when(i == 0)
        def _():
            w12[...] = jnp.dot(w1_ref[...], w2_ref[...],
                               preferred_element_type=jnp.float32)
            crow_ref[...] = jnp.dot(b1_ref[...], w2_ref[...],
                                    preferred_element_type=jnp.float32)

        d = deg_ref[0, :, 0:1] + deg_ref[1, :, 0:1] + 1.0
        d = jnp.maximum(d, 1e-12)
        dinv = lax.rsqrt(d)
        t_ref[...] = jnp.dot(emb_ref[...], w12[...],
                             preferred_element_type=jnp.float32) * dinv
        dv_ref[...] = jnp.concatenate([dinv, 1.0 / d] + [dinv] * 14, axis=1)

    return pl.pallas_call(
        kern,
        grid=(NCORE, NB),
        in_specs=[
            pl.BlockSpec((RBLK, D), lambda c, i: (i, 0)),
            pl.BlockSpec((D, H), lambda c, i: (0, 0)),
            pl.BlockSpec((H, HALF), lambda c, i: (0, c)),
            pl.BlockSpec((8, H), lambda c, i: (0, 0)),
            pl.BlockSpec((NCORE, RBLK, HALF), lambda c, i: (0, i, 0)),
        ],
        out_specs=[
            pl.BlockSpec((RBLK, HALF), lambda c, i: (c * NB + i, 0)),
            pl.BlockSpec((8, HALF), lambda c, i: (0, c)),
            pl.BlockSpec((RBLK, 16), lambda c, i: (i, 0)),
        ],
        out_shape=[
            jax.ShapeDtypeStruct((NCORE * N, HALF), jnp.float32),
            jax.ShapeDtypeStruct((8, D), jnp.float32),
            jax.ShapeDtypeStruct((N, 16), jnp.float32),
        ],
        scratch_shapes=[pltpu.VMEM((D, HALF), jnp.float32)],
    )(emb, W1, W2, b1p, degp)


def _tc_combine(s1, t, dinvs, crow):
    def kern(s1_ref, t_ref, dv_ref, crow_ref, t2_ref):
        dinv = dv_ref[:, 0:1]
        dinv2 = dv_ref[:, 1:2]
        t2_ref[...] = (s1_ref[...] + t_ref[...]) * dinv2 + crow_ref[0:1, :] * dinv

    return pl.pallas_call(
        kern,
        grid=(NCORE, NB),
        in_specs=[
            pl.BlockSpec((RBLK, HALF), lambda c, i: (c * NB + i, 0)),
            pl.BlockSpec((RBLK, HALF), lambda c, i: (c * NB + i, 0)),
            pl.BlockSpec((RBLK, 16), lambda c, i: (i, 0)),
            pl.BlockSpec((8, HALF), lambda c, i: (0, c)),
        ],
        out_specs=pl.BlockSpec((RBLK, HALF), lambda c, i: (c * NB + i, 0)),
        out_shape=jax.ShapeDtypeStruct((NCORE * N, HALF), jnp.float32),
    )(s1, t, dinvs, crow)


def _tc_finalize(s2, t2, dinvs):
    def kern(s2_ref, t2_ref, dv_ref, y_ref, st_ref, acc):
        i = pl.program_id(1)
        dinv = dv_ref[:, 0:1]
        y = (s2_ref[...] + t2_ref[...]) * dinv
        y_ref[...] = y

        @pl.when(i == 0)
        def _():
            acc[...] = jnp.zeros_like(acc)

        acc[0:1, :] += jnp.sum(y, axis=0, keepdims=True)
        acc[1:2, :] += jnp.sum(y * y, axis=0, keepdims=True)

        @pl.when(i == NB - 1)
        def _():
            st_ref[0, :, :] = acc[...]

    return pl.pallas_call(
        kern,
        grid=(NCORE, NB),
        in_specs=[
            pl.BlockSpec((RBLK, HALF), lambda c, i: (c * NB + i, 0)),
            pl.BlockSpec((RBLK, HALF), lambda c, i: (c * NB + i, 0)),
            pl.BlockSpec((RBLK, 16), lambda c, i: (i, 0)),
        ],
        out_specs=[
            pl.BlockSpec((RBLK, HALF), lambda c, i: (i, c)),
            pl.BlockSpec((1, 8, HALF), lambda c, i: (c, 0, 0)),
        ],
        out_shape=[
            jax.ShapeDtypeStruct((N, D), jnp.float32),
            jax.ShapeDtypeStruct((NCORE, 8, HALF), jnp.float32),
        ],
        scratch_shapes=[pltpu.VMEM((8, HALF), jnp.float32)],
    )(s2, t2, dinvs)


def _tc_batchnorm(y2, stats, gamma, beta):
    def kern(y_ref, st_ref, g_ref, b_ref, out_ref):
        sums = jnp.concatenate([st_ref[0, 0:1, :], st_ref[1, 0:1, :]], axis=1)
        sqs = jnp.concatenate([st_ref[0, 1:2, :], st_ref[1, 1:2, :]], axis=1)
        mu = sums * (1.0 / N)
        var = sqs * (1.0 / N) - mu * mu
        scale = g_ref[...] * lax.rsqrt(var + 1e-5)
        out_ref[...] = y_ref[...] * scale + (b_ref[...] - mu * scale)

    return pl.pallas_call(
        kern,
        grid=(NB,),
        in_specs=[
            pl.BlockSpec((RBLK, D), lambda i: (i, 0)),
            pl.BlockSpec((NCORE, 8, HALF), lambda i: (0, 0, 0)),
            pl.BlockSpec((1, D), lambda i: (0, 0)),
            pl.BlockSpec((1, D), lambda i: (0, 0)),
        ],
        out_specs=pl.BlockSpec((RBLK, D), lambda i: (i, 0)),
        out_shape=jax.ShapeDtypeStruct((N, D), jnp.float32),
    )(y2, stats, gamma, beta)


def kernel(edge_index, emb, W1, b1, W2, b2, gamma, beta):
    del b2
    src = edge_index[0]
    dst = edge_index[1]
    zeros128 = jnp.zeros((RPS, HALF), jnp.float32)
    ones128 = jnp.ones((CHUNK, HALF), jnp.float32)
    b1p = jnp.broadcast_to(b1[None, :], (8, H))

    degp = _sc_degree(dst, zeros128, ones128)
    t, crow, dinvs = _tc_matmul_scale(emb, W1, W2, b1p, degp)
    s1 = _sc_propagate(t, src, dst, zeros128)
    t2 = _tc_combine(s1, t, dinvs, crow)
    s2 = _sc_propagate(t2, src, dst, zeros128)
    y2, stats = _tc_finalize(s2, t2, dinvs)
    return _tc_batchnorm(y2, stats, gamma[None, :], beta[None, :])


# speedup vs baseline: 19.1995x; 1.0023x over previous
import functools

import jax
import jax.numpy as jnp
from jax import lax
from jax.experimental import pallas as pl
from jax.experimental.pallas import tpu as pltpu
from jax.experimental.pallas import tpu_sc as plsc

N = 10000
E = 160000
D = 256
H = 512
HALF = 128
NCORE = 2
NSUB = 16
RPS = 624
TAIL = 16
TAIL_OFF = NSUB * RPS
CHUNK = 128
NCHUNKS = E // CHUNK
RBLK = 1000
NB = N // RBLK

_MESH = dict(mesh=plsc.VectorSubcoreMesh(core_axis_name="c", subcore_axis_name="s"))


DKC = 39
DKMAX = DKC + 1


def _sc_degree(dst, zeros128, ones128):
    @functools.partial(
        pl.kernel,
        out_type=jax.ShapeDtypeStruct((NCORE, N, HALF), jnp.float32),
        scratch_types=[
            pltpu.VMEM_SHARED((N, HALF), jnp.float32),
            pltpu.VMEM((DKMAX * CHUNK,), jnp.int32),
            pltpu.VMEM((CHUNK,), jnp.int32),
            pltpu.VMEM((CHUNK,), jnp.int32),
            pltpu.VMEM((CHUNK, HALF), jnp.float32),
            pltpu.SemaphoreType.DMA,
            pltpu.SemaphoreType.DMA,
        ],
        **_MESH,
    )
    def deg_kernel(dst_ref, z_ref, ones_ref, out_ref,
                   acc_sh, dall, di0, di1, ones_v, sem0, sem1):
        c = lax.axis_index("c")
        s = lax.axis_index("s")
        w = c * NSUB + s
        base_e = w * (DKC * CHUNK)
        pltpu.sync_copy(dst_ref.at[pl.ds(base_e, DKC * CHUNK)],
                        dall.at[pl.ds(0, DKC * CHUNK)])

        @pl.when(w < 2)
        def _():
            pltpu.sync_copy(
                dst_ref.at[pl.ds(32 * DKC * CHUNK + w * CHUNK, CHUNK)],
                dall.at[pl.ds(DKC * CHUNK, CHUNK)])

        pltpu.sync_copy(ones_ref, ones_v)
        pltpu.sync_copy(z_ref, acc_sh.at[pl.ds(s * RPS, RPS)])

        @pl.when(s == NSUB - 1)
        def _():
            pltpu.sync_copy(z_ref.at[pl.ds(0, TAIL)],
                            acc_sh.at[pl.ds(TAIL_OFF, TAIL)])

        plsc.subcore_barrier()
        nk = jnp.where(w < 2, DKMAX, DKC)

        def body(k, carry):
            for g in range(CHUNK // 16):
                di0[pl.ds(g * 16, 16)] = dall[pl.ds(k * CHUNK + g * 16, 16)]
            pltpu.sync_copy(ones_v, acc_sh.at[di0], add=True)
            return carry

        lax.fori_loop(0, nk, body, 0)
        plsc.subcore_barrier()
        pltpu.sync_copy(acc_sh.at[pl.ds(s * RPS, RPS)],
                        out_ref.at[c, pl.ds(s * RPS, RPS)])

        @pl.when(s == NSUB - 1)
        def _():
            pltpu.sync_copy(acc_sh.at[pl.ds(TAIL_OFF, TAIL)],
                            out_ref.at[c, pl.ds(TAIL_OFF, TAIL)])

    return deg_kernel(dst, zeros128, ones128)


PCHUNK = 64
NKC = 156
NKMAX = NKC + 4


def _sc_propagate(t_flat, src, dst, zeros128):
    @functools.partial(
        pl.kernel,
        out_type=jax.ShapeDtypeStruct((NCORE * N, HALF), jnp.float32),
        scratch_types=[
            pltpu.VMEM_SHARED((N, HALF), jnp.float32),
            pltpu.VMEM((NKMAX * PCHUNK,), jnp.int32),
            pltpu.VMEM((NKMAX * PCHUNK,), jnp.int32),
            pltpu.VMEM((PCHUNK,), jnp.int32),
            pltpu.VMEM((PCHUNK,), jnp.int32),
            pltpu.VMEM((PCHUNK,), jnp.int32),
            pltpu.VMEM((PCHUNK,), jnp.int32),
            pltpu.VMEM((PCHUNK,), jnp.int32),
            pltpu.VMEM((PCHUNK,), jnp.int32),
            pltpu.VMEM((PCHUNK, HALF), jnp.float32),
            pltpu.VMEM((PCHUNK, HALF), jnp.float32),
            pltpu.VMEM((PCHUNK, HALF), jnp.float32),
            pltpu.SemaphoreType.DMA,
            pltpu.SemaphoreType.DMA,
            pltpu.SemaphoreType.DMA,
        ],
        **_MESH,
    )
    def mp_kernel(t_ref, src_ref, dst_ref, z_ref, out_ref,
                  acc_sh, sall, dall, si0, di0, si1, di1, si2, di2,
                  r0, r1, r2, sem0, sem1, sem2):
        c = lax.axis_index("c")
        s = lax.axis_index("s")
        base_e = s * (NKC * PCHUNK)
        pltpu.sync_copy(src_ref.at[pl.ds(base_e, NKC * PCHUNK)],
                        sall.at[pl.ds(0, NKC * PCHUNK)])
        pltpu.sync_copy(dst_ref.at[pl.ds(base_e, NKC * PCHUNK)],
                        dall.at[pl.ds(0, NKC * PCHUNK)])

        @pl.when(s == NSUB - 1)
        def _():
            pltpu.sync_copy(src_ref.at[pl.ds(NSUB * NKC * PCHUNK, 4 * PCHUNK)],
                            sall.at[pl.ds(NKC * PCHUNK, 4 * PCHUNK)])
            pltpu.sync_copy(dst_ref.at[pl.ds(NSUB * NKC * PCHUNK, 4 * PCHUNK)],
                            dall.at[pl.ds(NKC * PCHUNK, 4 * PCHUNK)])

        pltpu.sync_copy(z_ref, acc_sh.at[pl.ds(s * RPS, RPS)])

        @pl.when(s == NSUB - 1)
        def _():
            pltpu.sync_copy(z_ref.at[pl.ds(0, TAIL)],
                            acc_sh.at[pl.ds(TAIL_OFF, TAIL)])

        plsc.subcore_barrier()
        c_off = c * N
        nk = jnp.where(s == NSUB - 1, NKMAX, NKC)
        bufs = ((si0, di0, r0, sem0), (si1, di1, r1, sem1), (si2, di2, r2, sem2))

        def stage_idx(k, sidx, didx):
            for g in range(PCHUNK // 16):
                sl = pl.ds(k * PCHUNK + g * 16, 16)
                sidx[pl.ds(g * 16, 16)] = sall[sl] + c_off
                didx[pl.ds(g * 16, 16)] = dall[sl]

        def start(k, b):
            sidx, didx, rows, sem = bufs[b]
            stage_idx(k, sidx, didx)
            pltpu.async_copy(t_ref.at[sidx], rows, sem)

        start(0, 0)
        start(1, 1)

        def body(k, carry):
            for b in range(3):
                @pl.when(k % 3 == b)
                def _():
                    sidx, didx, rows, sem = bufs[b]

                    @pl.when(k + 2 < nk)
                    def _():
                        start(k + 2, (b + 2) % 3)

                    pltpu.make_async_copy(t_ref.at[sidx], rows, sem).wait()
                    pltpu.sync_copy(rows, acc_sh.at[didx], add=True)

            return carry

        lax.fori_loop(0, nk, body, 0)
        plsc.subcore_barrier()
        pltpu.sync_copy(acc_sh.at[pl.ds(s * RPS, RPS)],
                        out_ref.at[pl.ds(c * N + s * RPS, RPS)])

        @pl.when(s == NSUB - 1)
        def _():
            pltpu.sync_copy(acc_sh.at[pl.ds(TAIL_OFF, TAIL)],
                            out_ref.at[pl.ds(c * N + TAIL_OFF, TAIL)])

    return mp_kernel(t_flat, src, dst, zeros128)


def _tc_matmul_scale(emb, W1, W2, b1p, degp):
    def kern(emb_ref, w1_ref, w2_ref, b1_ref, deg_ref,
             t_ref, crow_ref, dv_ref, w12):
        i = pl.program_id(1)

        @pl.when(i == 0)
        def _():
            w12[...] = jnp.dot(w1_ref[...], w2_ref[...],
                               preferred_element_type=jnp.float32)
            crow_ref[...] = jnp.dot(b1_ref[...], w2_ref[...],
                                    preferred_element_type=jnp.float32)

        d = deg_ref[0, :, 0:1] + deg_ref[1, :, 0:1] + 1.0
        d = jnp.maximum(d, 1e-12)
        dinv = lax.rsqrt(d)
        t_ref[...] = jnp.dot(emb_ref[...], w12[...],
                             preferred_element_type=jnp.float32) * dinv
        dv_ref[...] = jnp.concatenate([dinv, 1.0 / d] + [dinv] * 14, axis=1)

    return pl.pallas_call(
        kern,
        grid=(NCORE, NB),
        in_specs=[
            pl.BlockSpec((RBLK, D), lambda c, i: (i, 0)),
            pl.BlockSpec((D, H), lambda c, i: (0, 0)),
            pl.BlockSpec((H, HALF), lambda c, i: (0, c)),
            pl.BlockSpec((8, H), lambda c, i: (0, 0)),
            pl.BlockSpec((NCORE, RBLK, HALF), lambda c, i: (0, i, 0)),
        ],
        out_specs=[
            pl.BlockSpec((RBLK, HALF), lambda c, i: (c * NB + i, 0)),
            pl.BlockSpec((8, HALF), lambda c, i: (0, c)),
            pl.BlockSpec((RBLK, 16), lambda c, i: (i, 0)),
        ],
        out_shape=[
            jax.ShapeDtypeStruct((NCORE * N, HALF), jnp.float32),
            jax.ShapeDtypeStruct((8, D), jnp.float32),
            jax.ShapeDtypeStruct((N, 16), jnp.float32),
        ],
        scratch_shapes=[pltpu.VMEM((D, HALF), jnp.float32)],
    )(emb, W1, W2, b1p, degp)


def _tc_combine(s1, t, dinvs, crow):
    def kern(s1_ref, t_ref, dv_ref, crow_ref, t2_ref):
        dinv = dv_ref[:, 0:1]
        dinv2 = dv_ref[:, 1:2]
        t2_ref[...] = (s1_ref[...] + t_ref[...]) * dinv2 + crow_ref[0:1, :] * dinv

    return pl.pallas_call(
        kern,
        grid=(NCORE, NB),
        in_specs=[
            pl.BlockSpec((RBLK, HALF), lambda c, i: (c * NB + i, 0)),
            pl.BlockSpec((RBLK, HALF), lambda c, i: (c * NB + i, 0)),
            pl.BlockSpec((RBLK, 16), lambda c, i: (i, 0)),
            pl.BlockSpec((8, HALF), lambda c, i: (0, c)),
        ],
        out_specs=pl.BlockSpec((RBLK, HALF), lambda c, i: (c * NB + i, 0)),
        out_shape=jax.ShapeDtypeStruct((NCORE * N, HALF), jnp.float32),
    )(s1, t, dinvs, crow)


def _tc_finalize_bn(s2, t2, dinvs, gamma, beta):
    def kern(s2_ref, t2_ref, dv_ref, g_ref, b_ref, out_ref, acc):
        p = pl.program_id(0)
        i = pl.program_id(1)

        @pl.when(p < 2)
        def _():
            dinv = dv_ref[:, 0:1]
            y = (s2_ref[...] + t2_ref[...]) * dinv
            out_ref[...] = y

            @pl.when((p == 0) & (i == 0))
            def _():
                acc[...] = jnp.zeros_like(acc)

            @pl.when(p == 0)
            def _():
                acc[0:1, :] += jnp.sum(y, axis=0, keepdims=True)
                acc[1:2, :] += jnp.sum(y * y, axis=0, keepdims=True)

            @pl.when(p == 1)
            def _():
                acc[2:3, :] += jnp.sum(y, axis=0, keepdims=True)
                acc[3:4, :] += jnp.sum(y * y, axis=0, keepdims=True)

        def norm(srow):
            mu = acc[srow:srow + 1, :] * (1.0 / N)
            var = acc[srow + 1:srow + 2, :] * (1.0 / N) - mu * mu
            scale = g_ref[...] * lax.rsqrt(var + 1e-5)
            out_ref[...] = out_ref[...] * scale + (b_ref[...] - mu * scale)

        @pl.when(p == 2)
        def _():
            norm(0)

        @pl.when(p == 3)
        def _():
            norm(2)

    return pl.pallas_call(
        kern,
        grid=(4, NB),
        in_specs=[
            pl.BlockSpec((RBLK, HALF),
                         lambda p, i: (jnp.where(p < 2, (p % 2) * NB + i, 0), 0)),
            pl.BlockSpec((RBLK, HALF),
                         lambda p, i: (jnp.where(p < 2, (p % 2) * NB + i, 0), 0)),
            pl.BlockSpec((RBLK, 16), lambda p, i: (jnp.where(p < 2, i, 0), 0)),
            pl.BlockSpec((1, HALF), lambda p, i: (0, p % 2)),
            pl.BlockSpec((1, HALF), lambda p, i: (0, p % 2)),
        ],
        out_specs=pl.BlockSpec((RBLK, HALF), lambda p, i: (i, p % 2)),
        out_shape=jax.ShapeDtypeStruct((N, D), jnp.float32),
        scratch_shapes=[pltpu.VMEM((8, HALF), jnp.float32)],
    )(s2, t2, dinvs, gamma, beta)


def kernel(edge_index, emb, W1, b1, W2, b2, gamma, beta):
    del b2
    src = edge_index[0]
    dst = edge_index[1]
    zeros128 = jnp.zeros((RPS, HALF), jnp.float32)
    ones128 = jnp.ones((CHUNK, HALF), jnp.float32)
    b1p = jnp.broadcast_to(b1[None, :], (8, H))

    degp = _sc_degree(dst, zeros128, ones128)
    t, crow, dinvs = _tc_matmul_scale(emb, W1, W2, b1p, degp)
    s1 = _sc_propagate(t, src, dst, zeros128)
    t2 = _tc_combine(s1, t, dinvs, crow)
    s2 = _sc_propagate(t2, src, dst, zeros128)
    return _tc_finalize_bn(s2, t2, dinvs, gamma[None, :], beta[None, :])


# speedup vs baseline: 19.3621x; 1.0085x over previous
import functools

import jax
import jax.numpy as jnp
from jax import lax
from jax.experimental import pallas as pl
from jax.experimental.pallas import tpu as pltpu
from jax.experimental.pallas import tpu_sc as plsc

N = 10000
E = 160000
D = 256
H = 512
HALF = 128
NCORE = 2
NSUB = 16
RPS = 624
TAIL = 16
TAIL_OFF = NSUB * RPS
CHUNK = 128
NCHUNKS = E // CHUNK
RBLK = 1000
NB = N // RBLK

_MESH = dict(mesh=plsc.VectorSubcoreMesh(core_axis_name="c", subcore_axis_name="s"))


DKC = 39
DKMAX = DKC + 1


def _sc_degree(dst, zeros128, ones128):
    @functools.partial(
        pl.kernel,
        out_type=jax.ShapeDtypeStruct((NCORE, N, HALF), jnp.float32),
        scratch_types=[
            pltpu.VMEM_SHARED((N, HALF), jnp.float32),
            pltpu.VMEM((DKMAX * CHUNK,), jnp.int32),
            pltpu.VMEM((CHUNK,), jnp.int32),
            pltpu.VMEM((CHUNK,), jnp.int32),
            pltpu.VMEM((CHUNK, HALF), jnp.float32),
            pltpu.SemaphoreType.DMA,
            pltpu.SemaphoreType.DMA,
        ],
        **_MESH,
    )
    def deg_kernel(dst_ref, z_ref, ones_ref, out_ref,
                   acc_sh, dall, di0, di1, ones_v, sem0, sem1):
        c = lax.axis_index("c")
        s = lax.axis_index("s")
        w = c * NSUB + s
        base_e = w * (DKC * CHUNK)
        pltpu.sync_copy(dst_ref.at[pl.ds(base_e, DKC * CHUNK)],
                        dall.at[pl.ds(0, DKC * CHUNK)])

        @pl.when(w < 2)
        def _():
            pltpu.sync_copy(
                dst_ref.at[pl.ds(32 * DKC * CHUNK + w * CHUNK, CHUNK)],
                dall.at[pl.ds(DKC * CHUNK, CHUNK)])

        pltpu.sync_copy(ones_ref, ones_v)
        pltpu.sync_copy(z_ref, acc_sh.at[pl.ds(s * RPS, RPS)])

        @pl.when(s == NSUB - 1)
        def _():
            pltpu.sync_copy(z_ref.at[pl.ds(0, TAIL)],
                            acc_sh.at[pl.ds(TAIL_OFF, TAIL)])

        plsc.subcore_barrier()
        nk = jnp.where(w < 2, DKMAX, DKC)

        def body(k, carry):
            for g in range(CHUNK // 16):
                di0[pl.ds(g * 16, 16)] = dall[pl.ds(k * CHUNK + g * 16, 16)]
            pltpu.sync_copy(ones_v, acc_sh.at[di0], add=True)
            return carry

        lax.fori_loop(0, nk, body, 0)
        plsc.subcore_barrier()
        pltpu.sync_copy(acc_sh.at[pl.ds(s * RPS, RPS)],
                        out_ref.at[c, pl.ds(s * RPS, RPS)])

        @pl.when(s == NSUB - 1)
        def _():
            pltpu.sync_copy(acc_sh.at[pl.ds(TAIL_OFF, TAIL)],
                            out_ref.at[c, pl.ds(TAIL_OFF, TAIL)])

    return deg_kernel(dst, zeros128, ones128)


PCHUNK = 64
NKC = 156
NKMAX = NKC + 4


def _sc_propagate(t_flat, src, dst, zeros128):
    @functools.partial(
        pl.kernel,
        out_type=jax.ShapeDtypeStruct((NCORE * N, HALF), jnp.float32),
        scratch_types=[
            pltpu.VMEM_SHARED((N, HALF), jnp.float32),
            pltpu.VMEM((NKMAX * PCHUNK,), jnp.int32),
            pltpu.VMEM((NKMAX * PCHUNK,), jnp.int32),
            pltpu.VMEM((PCHUNK,), jnp.int32),
            pltpu.VMEM((PCHUNK,), jnp.int32),
            pltpu.VMEM((PCHUNK,), jnp.int32),
            pltpu.VMEM((PCHUNK,), jnp.int32),
            pltpu.VMEM((PCHUNK,), jnp.int32),
            pltpu.VMEM((PCHUNK,), jnp.int32),
            pltpu.VMEM((PCHUNK, HALF), jnp.float32),
            pltpu.VMEM((PCHUNK, HALF), jnp.float32),
            pltpu.VMEM((PCHUNK, HALF), jnp.float32),
            pltpu.SemaphoreType.DMA,
            pltpu.SemaphoreType.DMA,
            pltpu.SemaphoreType.DMA,
        ],
        **_MESH,
    )
    def mp_kernel(t_ref, src_ref, dst_ref, z_ref, out_ref,
                  acc_sh, sall, dall, si0, di0, si1, di1, si2, di2,
                  r0, r1, r2, sem0, sem1, sem2):
        c = lax.axis_index("c")
        s = lax.axis_index("s")
        base_e = s * (NKC * PCHUNK)
        pltpu.sync_copy(src_ref.at[pl.ds(base_e, NKC * PCHUNK)],
                        sall.at[pl.ds(0, NKC * PCHUNK)])
        pltpu.sync_copy(dst_ref.at[pl.ds(base_e, NKC * PCHUNK)],
                        dall.at[pl.ds(0, NKC * PCHUNK)])

        @pl.when(s == NSUB - 1)
        def _():
            pltpu.sync_copy(src_ref.at[pl.ds(NSUB * NKC * PCHUNK, 4 * PCHUNK)],
                            sall.at[pl.ds(NKC * PCHUNK, 4 * PCHUNK)])
            pltpu.sync_copy(dst_ref.at[pl.ds(NSUB * NKC * PCHUNK, 4 * PCHUNK)],
                            dall.at[pl.ds(NKC * PCHUNK, 4 * PCHUNK)])

        pltpu.sync_copy(z_ref, acc_sh.at[pl.ds(s * RPS, RPS)])

        @pl.when(s == NSUB - 1)
        def _():
            pltpu.sync_copy(z_ref.at[pl.ds(0, TAIL)],
                            acc_sh.at[pl.ds(TAIL_OFF, TAIL)])

        plsc.subcore_barrier()
        c_off = c * N
        nk = jnp.where(s == NSUB - 1, NKMAX, NKC)
        bufs = ((si0, di0, r0, sem0), (si1, di1, r1, sem1), (si2, di2, r2, sem2))

        def stage_idx(k, sidx, didx):
            for g in range(PCHUNK // 16):
                sl = pl.ds(k * PCHUNK + g * 16, 16)
                sidx[pl.ds(g * 16, 16)] = sall[sl] + c_off
                didx[pl.ds(g * 16, 16)] = dall[sl]

        def start(k, b):
            sidx, didx, rows, sem = bufs[b]
            stage_idx(k, sidx, didx)
            pltpu.async_copy(t_ref.at[sidx], rows, sem)

        start(0, 0)
        start(1, 1)

        def body(k, carry):
            for b in range(3):
                @pl.when(k % 3 == b)
                def _():
                    sidx, didx, rows, sem = bufs[b]

                    @pl.when(k + 2 < nk)
                    def _():
                        start(k + 2, (b + 2) % 3)

                    pltpu.make_async_copy(t_ref.at[sidx], rows, sem).wait()
                    pltpu.sync_copy(rows, acc_sh.at[didx], add=True)

            return carry

        lax.fori_loop(0, nk, body, 0)
        plsc.subcore_barrier()
        pltpu.sync_copy(acc_sh.at[pl.ds(s * RPS, RPS)],
                        out_ref.at[pl.ds(c * N + s * RPS, RPS)])

        @pl.when(s == NSUB - 1)
        def _():
            pltpu.sync_copy(acc_sh.at[pl.ds(TAIL_OFF, TAIL)],
                            out_ref.at[pl.ds(c * N + TAIL_OFF, TAIL)])

    return mp_kernel(t_flat, src, dst, zeros128)


def _tc_matmul_scale(emb, W1, W2, b1p, degp):
    def kern(emb_ref, w1_ref, w2_ref, b1_ref, deg_ref,
             t_ref, crow_ref, dv_ref, w12):
        i = pl.program_id(1)

        @pl.when(i == 0)
        def _():
            w12[...] = jnp.dot(w1_ref[...], w2_ref[...],
                               preferred_element_type=jnp.float32)
            crow_ref[...] = jnp.dot(b1_ref[...], w2_ref[...],
                                    preferred_element_type=jnp.float32)

        d = deg_ref[0, :, 0:1] + deg_ref[1, :, 0:1] + 1.0
        d = jnp.maximum(d, 1e-12)
        dinv = lax.rsqrt(d)
        t_ref[...] = jnp.dot(emb_ref[...], w12[...],
                             preferred_element_type=jnp.float32) * dinv
        dv_ref[...] = jnp.concatenate([dinv, 1.0 / d] + [dinv] * 14, axis=1)

    return pl.pallas_call(
        kern,
        grid=(NCORE, NB),
        in_specs=[
            pl.BlockSpec((RBLK, D), lambda c, i: (i, 0)),
            pl.BlockSpec((D, H), lambda c, i: (0, 0)),
            pl.BlockSpec((H, HALF), lambda c, i: (0, c)),
            pl.BlockSpec((8, H), lambda c, i: (0, 0)),
            pl.BlockSpec((NCORE, RBLK, HALF), lambda c, i: (0, i, 0)),
        ],
        out_specs=[
            pl.BlockSpec((RBLK, HALF), lambda c, i: (c * NB + i, 0)),
            pl.BlockSpec((8, HALF), lambda c, i: (0, c)),
            pl.BlockSpec((RBLK, 16), lambda c, i: (i, 0)),
        ],
        out_shape=[
            jax.ShapeDtypeStruct((NCORE * N, HALF), jnp.float32),
            jax.ShapeDtypeStruct((8, D), jnp.float32),
            jax.ShapeDtypeStruct((N, 16), jnp.float32),
        ],
        scratch_shapes=[pltpu.VMEM((D, HALF), jnp.float32)],
    )(emb, W1, W2, b1p, degp)


def _tc_combine(s1, t, dinvs, crow):
    def kern(s1_ref, t_ref, dv_ref, crow_ref, t2_ref):
        dinv = dv_ref[:, 0:1]
        dinv2 = dv_ref[:, 1:2]
        t2_ref[...] = (s1_ref[...] + t_ref[...]) * dinv2 + crow_ref[0:1, :] * dinv

    return pl.pallas_call(
        kern,
        grid=(NCORE, NB),
        in_specs=[
            pl.BlockSpec((RBLK, HALF), lambda c, i: (c * NB + i, 0)),
            pl.BlockSpec((RBLK, HALF), lambda c, i: (c * NB + i, 0)),
            pl.BlockSpec((RBLK, 16), lambda c, i: (i, 0)),
            pl.BlockSpec((8, HALF), lambda c, i: (0, c)),
        ],
        out_specs=pl.BlockSpec((RBLK, HALF), lambda c, i: (c * NB + i, 0)),
        out_shape=jax.ShapeDtypeStruct((NCORE * N, HALF), jnp.float32),
    )(s1, t, dinvs, crow)


def _tc_finalize_bn(s2, t2, dinvs, gamma, beta):
    def kern(s2_ref, t2_ref, dv_ref, g_ref, b_ref, out_ref, ybuf, acc):
        p = pl.program_id(0)
        i = pl.program_id(1)
        rows = pl.ds(i * RBLK, RBLK)

        @pl.when(p < 2)
        def _():
            dinv = dv_ref[:, 0:1]
            y = (s2_ref[...] + t2_ref[...]) * dinv

            @pl.when((p == 0) & (i == 0))
            def _():
                acc[...] = jnp.zeros_like(acc)

            @pl.when(p == 0)
            def _():
                ybuf[rows, 0:HALF] = y
                acc[0:1, :] += jnp.sum(y, axis=0, keepdims=True)
                acc[1:2, :] += jnp.sum(y * y, axis=0, keepdims=True)

            @pl.when(p == 1)
            def _():
                ybuf[rows, HALF:D] = y
                acc[2:3, :] += jnp.sum(y, axis=0, keepdims=True)
                acc[3:4, :] += jnp.sum(y * y, axis=0, keepdims=True)

        def norm(srow, c0):
            mu = acc[srow:srow + 1, :] * (1.0 / N)
            var = acc[srow + 1:srow + 2, :] * (1.0 / N) - mu * mu
            scale = g_ref[...] * lax.rsqrt(var + 1e-5)
            out_ref[...] = ybuf[rows, c0:c0 + HALF] * scale + (b_ref[...] - mu * scale)

        @pl.when(p == 2)
        def _():
            norm(0, 0)

        @pl.when(p == 3)
        def _():
            norm(2, HALF)

    return pl.pallas_call(
        kern,
        grid=(4, NB),
        in_specs=[
            pl.BlockSpec((RBLK, HALF),
                         lambda p, i: (jnp.where(p < 2, (p % 2) * NB + i, 0), 0)),
            pl.BlockSpec((RBLK, HALF),
                         lambda p, i: (jnp.where(p < 2, (p % 2) * NB + i, 0), 0)),
            pl.BlockSpec((RBLK, 16), lambda p, i: (jnp.where(p < 2, i, 0), 0)),
            pl.BlockSpec((1, HALF), lambda p, i: (0, p % 2)),
            pl.BlockSpec((1, HALF), lambda p, i: (0, p % 2)),
        ],
        out_specs=pl.BlockSpec((RBLK, HALF),
                               lambda p, i: (jnp.where(p < 2, 0, i), p % 2)),
        out_shape=jax.ShapeDtypeStruct((N, D), jnp.float32),
        scratch_shapes=[pltpu.VMEM((N, D), jnp.float32),
                        pltpu.VMEM((8, HALF), jnp.float32)],
    )(s2, t2, dinvs, gamma, beta)


def kernel(edge_index, emb, W1, b1, W2, b2, gamma, beta):
    del b2
    src = edge_index[0]
    dst = edge_index[1]
    zeros128 = jnp.zeros((RPS, HALF), jnp.float32)
    ones128 = jnp.ones((CHUNK, HALF), jnp.float32)
    b1p = jnp.broadcast_to(b1[None, :], (8, H))

    degp = _sc_degree(dst, zeros128, ones128)
    t, crow, dinvs = _tc_matmul_scale(emb, W1, W2, b1p, degp)
    s1 = _sc_propagate(t, src, dst, zeros128)
    t2 = _tc_combine(s1, t, dinvs, crow)
    s2 = _sc_propagate(t2, src, dst, zeros128)
    return _tc_finalize_bn(s2, t2, dinvs, gamma[None, :], beta[None, :])


# speedup vs baseline: 19.5067x; 1.0075x over previous
import functools

import jax
import jax.numpy as jnp
from jax import lax
from jax.experimental import pallas as pl
from jax.experimental.pallas import tpu as pltpu
from jax.experimental.pallas import tpu_sc as plsc

N = 10000
E = 160000
D = 256
H = 512
HALF = 128
NCORE = 2
NSUB = 16
RPS = 624
TAIL = 16
TAIL_OFF = NSUB * RPS
CHUNK = 128
NCHUNKS = E // CHUNK
RBLK = 1000
NB = N // RBLK

_MESH = dict(mesh=plsc.VectorSubcoreMesh(core_axis_name="c", subcore_axis_name="s"))


DKC = 39
DKMAX = DKC + 1


def _sc_degree(dst, zeros128, ones128):
    @functools.partial(
        pl.kernel,
        out_type=jax.ShapeDtypeStruct((NCORE, N, HALF), jnp.float32),
        scratch_types=[
            pltpu.VMEM_SHARED((N, HALF), jnp.float32),
            pltpu.VMEM((DKMAX * CHUNK,), jnp.int32),
            pltpu.VMEM((CHUNK,), jnp.int32),
            pltpu.VMEM((CHUNK,), jnp.int32),
            pltpu.VMEM((CHUNK, HALF), jnp.float32),
            pltpu.SemaphoreType.DMA,
            pltpu.SemaphoreType.DMA,
        ],
        **_MESH,
    )
    def deg_kernel(dst_ref, z_ref, ones_ref, out_ref,
                   acc_sh, dall, di0, di1, ones_v, sem0, sem1):
        c = lax.axis_index("c")
        s = lax.axis_index("s")
        w = c * NSUB + s
        base_e = w * (DKC * CHUNK)
        pltpu.sync_copy(dst_ref.at[pl.ds(base_e, DKC * CHUNK)],
                        dall.at[pl.ds(0, DKC * CHUNK)])

        @pl.when(w < 2)
        def _():
            pltpu.sync_copy(
                dst_ref.at[pl.ds(32 * DKC * CHUNK + w * CHUNK, CHUNK)],
                dall.at[pl.ds(DKC * CHUNK, CHUNK)])

        pltpu.sync_copy(ones_ref, ones_v)
        pltpu.sync_copy(z_ref, acc_sh.at[pl.ds(s * RPS, RPS)])

        @pl.when(s == NSUB - 1)
        def _():
            pltpu.sync_copy(z_ref.at[pl.ds(0, TAIL)],
                            acc_sh.at[pl.ds(TAIL_OFF, TAIL)])

        plsc.subcore_barrier()
        nk = jnp.where(w < 2, DKMAX, DKC)

        def body(k, carry):
            for g in range(CHUNK // 16):
                di0[pl.ds(g * 16, 16)] = dall[pl.ds(k * CHUNK + g * 16, 16)]
            pltpu.sync_copy(ones_v, acc_sh.at[di0], add=True)
            return carry

        lax.fori_loop(0, nk, body, 0)
        plsc.subcore_barrier()
        pltpu.sync_copy(acc_sh.at[pl.ds(s * RPS, RPS)],
                        out_ref.at[c, pl.ds(s * RPS, RPS)])

        @pl.when(s == NSUB - 1)
        def _():
            pltpu.sync_copy(acc_sh.at[pl.ds(TAIL_OFF, TAIL)],
                            out_ref.at[c, pl.ds(TAIL_OFF, TAIL)])

    return deg_kernel(dst, zeros128, ones128)


PCHUNK = 64
NKC = 156
NKMAX = NKC + 4


def _sc_propagate(t_flat, src, dst, zeros128):
    @functools.partial(
        pl.kernel,
        out_type=jax.ShapeDtypeStruct((NCORE * N, HALF), jnp.float32),
        scratch_types=[
            pltpu.VMEM_SHARED((N, HALF), jnp.float32),
            pltpu.VMEM((NKMAX * PCHUNK,), jnp.int32),
            pltpu.VMEM((NKMAX * PCHUNK,), jnp.int32),
            pltpu.VMEM((PCHUNK,), jnp.int32),
            pltpu.VMEM((PCHUNK,), jnp.int32),
            pltpu.VMEM((PCHUNK,), jnp.int32),
            pltpu.VMEM((PCHUNK,), jnp.int32),
            pltpu.VMEM((PCHUNK,), jnp.int32),
            pltpu.VMEM((PCHUNK,), jnp.int32),
            pltpu.VMEM((PCHUNK, HALF), jnp.float32),
            pltpu.VMEM((PCHUNK, HALF), jnp.float32),
            pltpu.VMEM((PCHUNK, HALF), jnp.float32),
            pltpu.SemaphoreType.DMA,
            pltpu.SemaphoreType.DMA,
            pltpu.SemaphoreType.DMA,
        ],
        **_MESH,
    )
    def mp_kernel(t_ref, src_ref, dst_ref, z_ref, out_ref,
                  acc_sh, sall, dall, si0, di0, si1, di1, si2, di2,
                  r0, r1, r2, sem0, sem1, sem2):
        c = lax.axis_index("c")
        s = lax.axis_index("s")
        base_e = s * (NKC * PCHUNK)
        pltpu.sync_copy(src_ref.at[pl.ds(base_e, NKC * PCHUNK)],
                        sall.at[pl.ds(0, NKC * PCHUNK)])
        pltpu.sync_copy(dst_ref.at[pl.ds(base_e, NKC * PCHUNK)],
                        dall.at[pl.ds(0, NKC * PCHUNK)])

        @pl.when(s == NSUB - 1)
        def _():
            pltpu.sync_copy(src_ref.at[pl.ds(NSUB * NKC * PCHUNK, 4 * PCHUNK)],
                            sall.at[pl.ds(NKC * PCHUNK, 4 * PCHUNK)])
            pltpu.sync_copy(dst_ref.at[pl.ds(NSUB * NKC * PCHUNK, 4 * PCHUNK)],
                            dall.at[pl.ds(NKC * PCHUNK, 4 * PCHUNK)])

        pltpu.sync_copy(z_ref, acc_sh.at[pl.ds(s * RPS, RPS)])

        @pl.when(s == NSUB - 1)
        def _():
            pltpu.sync_copy(z_ref.at[pl.ds(0, TAIL)],
                            acc_sh.at[pl.ds(TAIL_OFF, TAIL)])

        plsc.subcore_barrier()
        c_off = c * N
        nk = jnp.where(s == NSUB - 1, NKMAX, NKC)
        bufs = ((si0, di0, r0, sem0), (si1, di1, r1, sem1), (si2, di2, r2, sem2))

        def stage_idx(k, sidx, didx):
            for g in range(PCHUNK // 16):
                sl = pl.ds(k * PCHUNK + g * 16, 16)
                sidx[pl.ds(g * 16, 16)] = sall[sl] + c_off
                didx[pl.ds(g * 16, 16)] = dall[sl]

        def start(k, b):
            sidx, didx, rows, sem = bufs[b]
            stage_idx(k, sidx, didx)
            pltpu.async_copy(t_ref.at[sidx], rows, sem)

        start(0, 0)
        start(1, 1)

        def body(k, carry):
            for b in range(3):
                @pl.when(k % 3 == b)
                def _():
                    sidx, didx, rows, sem = bufs[b]

                    @pl.when(k + 2 < nk)
                    def _():
                        start(k + 2, (b + 2) % 3)

                    pltpu.make_async_copy(t_ref.at[sidx], rows, sem).wait()
                    pltpu.sync_copy(rows, acc_sh.at[didx], add=True)

            return carry

        lax.fori_loop(0, nk, body, 0)
        plsc.subcore_barrier()
        pltpu.sync_copy(acc_sh.at[pl.ds(s * RPS, RPS)],
                        out_ref.at[pl.ds(c * N + s * RPS, RPS)])

        @pl.when(s == NSUB - 1)
        def _():
            pltpu.sync_copy(acc_sh.at[pl.ds(TAIL_OFF, TAIL)],
                            out_ref.at[pl.ds(c * N + TAIL_OFF, TAIL)])

    return mp_kernel(t_flat, src, dst, zeros128)


def _tc_matmul(emb, W1, W2, b1p):
    def kern(emb_ref, w1_ref, w2_ref, b1_ref, mf_ref, crow_ref, w12):
        i = pl.program_id(1)

        @pl.when(i == 0)
        def _():
            w12[...] = jnp.dot(w1_ref[...], w2_ref[...],
                               preferred_element_type=jnp.float32)
            crow_ref[...] = jnp.dot(b1_ref[...], w2_ref[...],
                                    preferred_element_type=jnp.float32)

        mf_ref[...] = jnp.dot(emb_ref[...], w12[...],
                              preferred_element_type=jnp.float32)

    return pl.pallas_call(
        kern,
        grid=(NCORE, NB),
        in_specs=[
            pl.BlockSpec((RBLK, D), lambda c, i: (i, 0)),
            pl.BlockSpec((D, H), lambda c, i: (0, 0)),
            pl.BlockSpec((H, HALF), lambda c, i: (0, c)),
            pl.BlockSpec((8, H), lambda c, i: (0, 0)),
        ],
        out_specs=[
            pl.BlockSpec((RBLK, HALF), lambda c, i: (c * NB + i, 0)),
            pl.BlockSpec((8, HALF), lambda c, i: (0, c)),
        ],
        out_shape=[
            jax.ShapeDtypeStruct((NCORE * N, HALF), jnp.float32),
            jax.ShapeDtypeStruct((8, D), jnp.float32),
        ],
        scratch_shapes=[pltpu.VMEM((D, HALF), jnp.float32)],
    )(emb, W1, W2, b1p)


def _tc_scale(mf, degp):
    def kern(mf_ref, deg_ref, t_ref, dv_ref):
        d = deg_ref[0, :, 0:1] + deg_ref[1, :, 0:1] + 1.0
        d = jnp.maximum(d, 1e-12)
        dinv = lax.rsqrt(d)
        t_ref[...] = mf_ref[...] * dinv
        dv_ref[...] = jnp.concatenate([dinv, 1.0 / d] + [dinv] * 14, axis=1)

    return pl.pallas_call(
        kern,
        grid=(NCORE, NB),
        in_specs=[
            pl.BlockSpec((RBLK, HALF), lambda c, i: (c * NB + i, 0)),
            pl.BlockSpec((NCORE, RBLK, HALF), lambda c, i: (0, i, 0)),
        ],
        out_specs=[
            pl.BlockSpec((RBLK, HALF), lambda c, i: (c * NB + i, 0)),
            pl.BlockSpec((RBLK, 16), lambda c, i: (i, 0)),
        ],
        out_shape=[
            jax.ShapeDtypeStruct((NCORE * N, HALF), jnp.float32),
            jax.ShapeDtypeStruct((N, 16), jnp.float32),
        ],
    )(mf, degp)


def _tc_combine(s1, t, dinvs, crow):
    def kern(s1_ref, t_ref, dv_ref, crow_ref, t2_ref):
        dinv = dv_ref[:, 0:1]
        dinv2 = dv_ref[:, 1:2]
        t2_ref[...] = (s1_ref[...] + t_ref[...]) * dinv2 + crow_ref[0:1, :] * dinv

    return pl.pallas_call(
        kern,
        grid=(NCORE, NB),
        in_specs=[
            pl.BlockSpec((RBLK, HALF), lambda c, i: (c * NB + i, 0)),
            pl.BlockSpec((RBLK, HALF), lambda c, i: (c * NB + i, 0)),
            pl.BlockSpec((RBLK, 16), lambda c, i: (i, 0)),
            pl.BlockSpec((8, HALF), lambda c, i: (0, c)),
        ],
        out_specs=pl.BlockSpec((RBLK, HALF), lambda c, i: (c * NB + i, 0)),
        out_shape=jax.ShapeDtypeStruct((NCORE * N, HALF), jnp.float32),
    )(s1, t, dinvs, crow)


def _tc_finalize_bn(s2, t2, dinvs, gamma, beta):
    def kern(s2_ref, t2_ref, dv_ref, g_ref, b_ref, out_ref, ybuf, acc):
        p = pl.program_id(0)
        i = pl.program_id(1)
        rows = pl.ds(i * RBLK, RBLK)

        @pl.when(p < 2)
        def _():
            dinv = dv_ref[:, 0:1]
            y = (s2_ref[...] + t2_ref[...]) * dinv

            @pl.when((p == 0) & (i == 0))
            def _():
                acc[...] = jnp.zeros_like(acc)

            @pl.when(p == 0)
            def _():
                ybuf[rows, 0:HALF] = y
                acc[0:1, :] += jnp.sum(y, axis=0, keepdims=True)
                acc[1:2, :] += jnp.sum(y * y, axis=0, keepdims=True)

            @pl.when(p == 1)
            def _():
                ybuf[rows, HALF:D] = y
                acc[2:3, :] += jnp.sum(y, axis=0, keepdims=True)
                acc[3:4, :] += jnp.sum(y * y, axis=0, keepdims=True)

        def norm(srow, c0):
            mu = acc[srow:srow + 1, :] * (1.0 / N)
            var = acc[srow + 1:srow + 2, :] * (1.0 / N) - mu * mu
            scale = g_ref[...] * lax.rsqrt(var + 1e-5)
            out_ref[...] = ybuf[rows, c0:c0 + HALF] * scale + (b_ref[...] - mu * scale)

        @pl.when(p == 2)
        def _():
            norm(0, 0)

        @pl.when(p == 3)
        def _():
            norm(2, HALF)

    return pl.pallas_call(
        kern,
        grid=(4, NB),
        in_specs=[
            pl.BlockSpec((RBLK, HALF),
                         lambda p, i: (jnp.where(p < 2, (p % 2) * NB + i, 0), 0)),
            pl.BlockSpec((RBLK, HALF),
                         lambda p, i: (jnp.where(p < 2, (p % 2) * NB + i, 0), 0)),
            pl.BlockSpec((RBLK, 16), lambda p, i: (jnp.where(p < 2, i, 0), 0)),
            pl.BlockSpec((1, HALF), lambda p, i: (0, p % 2)),
            pl.BlockSpec((1, HALF), lambda p, i: (0, p % 2)),
        ],
        out_specs=pl.BlockSpec((RBLK, HALF),
                               lambda p, i: (jnp.where(p < 2, 0, i), p % 2)),
        out_shape=jax.ShapeDtypeStruct((N, D), jnp.float32),
        scratch_shapes=[pltpu.VMEM((N, D), jnp.float32),
                        pltpu.VMEM((8, HALF), jnp.float32)],
    )(s2, t2, dinvs, gamma, beta)


def kernel(edge_index, emb, W1, b1, W2, b2, gamma, beta):
    del b2
    src = edge_index[0]
    dst = edge_index[1]
    zeros128 = jnp.zeros((RPS, HALF), jnp.float32)
    ones128 = jnp.ones((CHUNK, HALF), jnp.float32)
    b1p = jnp.broadcast_to(b1[None, :], (8, H))

    degp = _sc_degree(dst, zeros128, ones128)
    mf, crow = _tc_matmul(emb, W1, W2, b1p)
    t, dinvs = _tc_scale(mf, degp)
    s1 = _sc_propagate(t, src, dst, zeros128)
    t2 = _tc_combine(s1, t, dinvs, crow)
    s2 = _sc_propagate(t2, src, dst, zeros128)
    return _tc_finalize_bn(s2, t2, dinvs, gamma[None, :], beta[None, :])


# speedup vs baseline: 19.5987x; 1.0047x over previous
import functools

import jax
import jax.numpy as jnp
from jax import lax
from jax.experimental import pallas as pl
from jax.experimental.pallas import tpu as pltpu
from jax.experimental.pallas import tpu_sc as plsc

N = 10000
E = 160000
D = 256
H = 512
HALF = 128
NCORE = 2
NSUB = 16
RPS = 624
TAIL = 16
TAIL_OFF = NSUB * RPS
CHUNK = 128
NCHUNKS = E // CHUNK
RBLK = 1000
NB = N // RBLK

_MESH = dict(mesh=plsc.VectorSubcoreMesh(core_axis_name="c", subcore_axis_name="s"))


DKC = 39
DKMAX = DKC + 1


def _sc_degree(dst, zeros128, ones128):
    @functools.partial(
        pl.kernel,
        out_type=jax.ShapeDtypeStruct((NCORE, N, HALF), jnp.float32),
        scratch_types=[
            pltpu.VMEM_SHARED((N, HALF), jnp.float32),
            pltpu.VMEM((DKMAX * CHUNK,), jnp.int32),
            pltpu.VMEM((CHUNK,), jnp.int32),
            pltpu.VMEM((CHUNK,), jnp.int32),
            pltpu.VMEM((CHUNK, HALF), jnp.float32),
            pltpu.SemaphoreType.DMA,
            pltpu.SemaphoreType.DMA,
        ],
        **_MESH,
    )
    def deg_kernel(dst_ref, z_ref, ones_ref, out_ref,
                   acc_sh, dall, di0, di1, ones_v, sem0, sem1):
        c = lax.axis_index("c")
        s = lax.axis_index("s")
        w = c * NSUB + s
        base_e = w * (DKC * CHUNK)
        pltpu.sync_copy(dst_ref.at[pl.ds(base_e, DKC * CHUNK)],
                        dall.at[pl.ds(0, DKC * CHUNK)])

        @pl.when(w < 2)
        def _():
            pltpu.sync_copy(
                dst_ref.at[pl.ds(32 * DKC * CHUNK + w * CHUNK, CHUNK)],
                dall.at[pl.ds(DKC * CHUNK, CHUNK)])

        pltpu.sync_copy(ones_ref, ones_v)
        pltpu.sync_copy(z_ref, acc_sh.at[pl.ds(s * RPS, RPS)])

        @pl.when(s == NSUB - 1)
        def _():
            pltpu.sync_copy(z_ref.at[pl.ds(0, TAIL)],
                            acc_sh.at[pl.ds(TAIL_OFF, TAIL)])

        plsc.subcore_barrier()
        nk = jnp.where(w < 2, DKMAX, DKC)

        def body(k, carry):
            for g in range(CHUNK // 16):
                di0[pl.ds(g * 16, 16)] = dall[pl.ds(k * CHUNK + g * 16, 16)]
            pltpu.sync_copy(ones_v, acc_sh.at[di0], add=True)
            return carry

        lax.fori_loop(0, nk, body, 0)
        plsc.subcore_barrier()
        pltpu.sync_copy(acc_sh.at[pl.ds(s * RPS, RPS)],
                        out_ref.at[c, pl.ds(s * RPS, RPS)])

        @pl.when(s == NSUB - 1)
        def _():
            pltpu.sync_copy(acc_sh.at[pl.ds(TAIL_OFF, TAIL)],
                            out_ref.at[c, pl.ds(TAIL_OFF, TAIL)])

    return deg_kernel(dst, zeros128, ones128)


PCHUNK = 64
NKC = 156
NKMAX = NKC + 4


def _sc_propagate(t_flat, src, dst, zeros128):
    @functools.partial(
        pl.kernel,
        out_type=jax.ShapeDtypeStruct((NCORE * N, HALF), jnp.float32),
        scratch_types=[
            pltpu.VMEM_SHARED((N, HALF), jnp.float32),
            pltpu.VMEM((NKMAX * PCHUNK,), jnp.int32),
            pltpu.VMEM((NKMAX * PCHUNK,), jnp.int32),
            pltpu.VMEM((PCHUNK,), jnp.int32),
            pltpu.VMEM((PCHUNK,), jnp.int32),
            pltpu.VMEM((PCHUNK,), jnp.int32),
            pltpu.VMEM((PCHUNK,), jnp.int32),
            pltpu.VMEM((PCHUNK,), jnp.int32),
            pltpu.VMEM((PCHUNK,), jnp.int32),
            pltpu.VMEM((PCHUNK, HALF), jnp.float32),
            pltpu.VMEM((PCHUNK, HALF), jnp.float32),
            pltpu.VMEM((PCHUNK, HALF), jnp.float32),
            pltpu.SemaphoreType.DMA,
            pltpu.SemaphoreType.DMA,
            pltpu.SemaphoreType.DMA,
        ],
        **_MESH,
    )
    def mp_kernel(t_ref, src_ref, dst_ref, z_ref, out_ref,
                  acc_sh, sall, dall, si0, di0, si1, di1, si2, di2,
                  r0, r1, r2, sem0, sem1, sem2):
        c = lax.axis_index("c")
        s = lax.axis_index("s")
        base_e = s * (NKC * PCHUNK)
        pltpu.sync_copy(src_ref.at[pl.ds(base_e, NKC * PCHUNK)],
                        sall.at[pl.ds(0, NKC * PCHUNK)])
        pltpu.sync_copy(dst_ref.at[pl.ds(base_e, NKC * PCHUNK)],
                        dall.at[pl.ds(0, NKC * PCHUNK)])

        @pl.when(s == NSUB - 1)
        def _():
            pltpu.sync_copy(src_ref.at[pl.ds(NSUB * NKC * PCHUNK, 4 * PCHUNK)],
                            sall.at[pl.ds(NKC * PCHUNK, 4 * PCHUNK)])
            pltpu.sync_copy(dst_ref.at[pl.ds(NSUB * NKC * PCHUNK, 4 * PCHUNK)],
                            dall.at[pl.ds(NKC * PCHUNK, 4 * PCHUNK)])

        c_off = c * N
        nk = jnp.where(s == NSUB - 1, NKMAX, NKC)
        bufs = ((si0, di0, r0, sem0), (si1, di1, r1, sem1), (si2, di2, r2, sem2))

        def stage_idx(k, sidx, didx):
            for g in range(PCHUNK // 16):
                sl = pl.ds(k * PCHUNK + g * 16, 16)
                sidx[pl.ds(g * 16, 16)] = sall[sl] + c_off
                didx[pl.ds(g * 16, 16)] = dall[sl]

        def start(k, b):
            sidx, didx, rows, sem = bufs[b]
            stage_idx(k, sidx, didx)
            pltpu.async_copy(t_ref.at[sidx], rows, sem)

        start(0, 0)
        start(1, 1)
        pltpu.sync_copy(z_ref, acc_sh.at[pl.ds(s * RPS, RPS)])

        @pl.when(s == NSUB - 1)
        def _():
            pltpu.sync_copy(z_ref.at[pl.ds(0, TAIL)],
                            acc_sh.at[pl.ds(TAIL_OFF, TAIL)])

        plsc.subcore_barrier()

        def body(k, carry):
            for b in range(3):
                @pl.when(k % 3 == b)
                def _():
                    sidx, didx, rows, sem = bufs[b]

                    @pl.when(k + 2 < nk)
                    def _():
                        start(k + 2, (b + 2) % 3)

                    pltpu.make_async_copy(t_ref.at[sidx], rows, sem).wait()
                    pltpu.sync_copy(rows, acc_sh.at[didx], add=True)

            return carry

        lax.fori_loop(0, nk, body, 0)
        plsc.subcore_barrier()
        pltpu.sync_copy(acc_sh.at[pl.ds(s * RPS, RPS)],
                        out_ref.at[pl.ds(c * N + s * RPS, RPS)])

        @pl.when(s == NSUB - 1)
        def _():
            pltpu.sync_copy(acc_sh.at[pl.ds(TAIL_OFF, TAIL)],
                            out_ref.at[pl.ds(c * N + TAIL_OFF, TAIL)])

    return mp_kernel(t_flat, src, dst, zeros128)


def _tc_matmul(emb, W1, W2, b1p):
    def kern(emb_ref, w1_ref, w2_ref, b1_ref, mf_ref, crow_ref, w12):
        i = pl.program_id(1)

        @pl.when(i == 0)
        def _():
            w12[...] = jnp.dot(w1_ref[...], w2_ref[...],
                               preferred_element_type=jnp.float32)
            crow_ref[...] = jnp.dot(b1_ref[...], w2_ref[...],
                                    preferred_element_type=jnp.float32)

        mf_ref[...] = jnp.dot(emb_ref[...], w12[...],
                              preferred_element_type=jnp.float32)

    return pl.pallas_call(
        kern,
        grid=(NCORE, NB),
        in_specs=[
            pl.BlockSpec((RBLK, D), lambda c, i: (i, 0)),
            pl.BlockSpec((D, H), lambda c, i: (0, 0)),
            pl.BlockSpec((H, HALF), lambda c, i: (0, c)),
            pl.BlockSpec((8, H), lambda c, i: (0, 0)),
        ],
        out_specs=[
            pl.BlockSpec((RBLK, HALF), lambda c, i: (c * NB + i, 0)),
            pl.BlockSpec((8, HALF), lambda c, i: (0, c)),
        ],
        out_shape=[
            jax.ShapeDtypeStruct((NCORE * N, HALF), jnp.float32),
            jax.ShapeDtypeStruct((8, D), jnp.float32),
        ],
        scratch_shapes=[pltpu.VMEM((D, HALF), jnp.float32)],
    )(emb, W1, W2, b1p)


def _tc_scale(mf, degp):
    def kern(mf_ref, deg_ref, t_ref, dv_ref):
        d = deg_ref[0, :, 0:1] + deg_ref[1, :, 0:1] + 1.0
        d = jnp.maximum(d, 1e-12)
        dinv = lax.rsqrt(d)
        t_ref[...] = mf_ref[...] * dinv
        dv_ref[...] = jnp.concatenate([dinv, 1.0 / d] + [dinv] * 14, axis=1)

    return pl.pallas_call(
        kern,
        grid=(NCORE, NB),
        in_specs=[
            pl.BlockSpec((RBLK, HALF), lambda c, i: (c * NB + i, 0)),
            pl.BlockSpec((NCORE, RBLK, HALF), lambda c, i: (0, i, 0)),
        ],
        out_specs=[
            pl.BlockSpec((RBLK, HALF), lambda c, i: (c * NB + i, 0)),
            pl.BlockSpec((RBLK, 16), lambda c, i: (i, 0)),
        ],
        out_shape=[
            jax.ShapeDtypeStruct((NCORE * N, HALF), jnp.float32),
            jax.ShapeDtypeStruct((N, 16), jnp.float32),
        ],
    )(mf, degp)


def _tc_combine(s1, t, dinvs, crow):
    def kern(s1_ref, t_ref, dv_ref, crow_ref, t2_ref):
        dinv = dv_ref[:, 0:1]
        dinv2 = dv_ref[:, 1:2]
        t2_ref[...] = (s1_ref[...] + t_ref[...]) * dinv2 + crow_ref[0:1, :] * dinv

    return pl.pallas_call(
        kern,
        grid=(NCORE, NB),
        in_specs=[
            pl.BlockSpec((RBLK, HALF), lambda c, i: (c * NB + i, 0)),
            pl.BlockSpec((RBLK, HALF), lambda c, i: (c * NB + i, 0)),
            pl.BlockSpec((RBLK, 16), lambda c, i: (i, 0)),
            pl.BlockSpec((8, HALF), lambda c, i: (0, c)),
        ],
        out_specs=pl.BlockSpec((RBLK, HALF), lambda c, i: (c * NB + i, 0)),
        out_shape=jax.ShapeDtypeStruct((NCORE * N, HALF), jnp.float32),
    )(s1, t, dinvs, crow)


def _tc_finalize_bn(s2, t2, dinvs, gamma, beta):
    def kern(s2_ref, t2_ref, dv_ref, g_ref, b_ref, out_ref, ybuf, acc):
        p = pl.program_id(0)
        i = pl.program_id(1)
        rows = pl.ds(i * RBLK, RBLK)

        @pl.when(p < 2)
        def _():
            dinv = dv_ref[:, 0:1]
            y = (s2_ref[...] + t2_ref[...]) * dinv

            @pl.when((p == 0) & (i == 0))
            def _():
                acc[...] = jnp.zeros_like(acc)

            @pl.when(p == 0)
            def _():
                ybuf[rows, 0:HALF] = y
                acc[0:1, :] += jnp.sum(y, axis=0, keepdims=True)
                acc[1:2, :] += jnp.sum(y * y, axis=0, keepdims=True)

            @pl.when(p == 1)
            def _():
                ybuf[rows, HALF:D] = y
                acc[2:3, :] += jnp.sum(y, axis=0, keepdims=True)
                acc[3:4, :] += jnp.sum(y * y, axis=0, keepdims=True)

        def norm(srow, c0):
            mu = acc[srow:srow + 1, :] * (1.0 / N)
            var = acc[srow + 1:srow + 2, :] * (1.0 / N) - mu * mu
            scale = g_ref[...] * lax.rsqrt(var + 1e-5)
            out_ref[...] = ybuf[rows, c0:c0 + HALF] * scale + (b_ref[...] - mu * scale)

        @pl.when(p == 2)
        def _():
            norm(0, 0)

        @pl.when(p == 3)
        def _():
            norm(2, HALF)

    return pl.pallas_call(
        kern,
        grid=(4, NB),
        in_specs=[
            pl.BlockSpec((RBLK, HALF),
                         lambda p, i: (jnp.where(p < 2, (p % 2) * NB + i, 0), 0)),
            pl.BlockSpec((RBLK, HALF),
                         lambda p, i: (jnp.where(p < 2, (p % 2) * NB + i, 0), 0)),
            pl.BlockSpec((RBLK, 16), lambda p, i: (jnp.where(p < 2, i, 0), 0)),
            pl.BlockSpec((1, HALF), lambda p, i: (0, p % 2)),
            pl.BlockSpec((1, HALF), lambda p, i: (0, p % 2)),
        ],
        out_specs=pl.BlockSpec((RBLK, HALF),
                               lambda p, i: (jnp.where(p < 2, 0, i), p % 2)),
        out_shape=jax.ShapeDtypeStruct((N, D), jnp.float32),
        scratch_shapes=[pltpu.VMEM((N, D), jnp.float32),
                        pltpu.VMEM((8, HALF), jnp.float32)],
    )(s2, t2, dinvs, gamma, beta)


def kernel(edge_index, emb, W1, b1, W2, b2, gamma, beta):
    del b2
    src = edge_index[0]
    dst = edge_index[1]
    zeros128 = jnp.zeros((RPS, HALF), jnp.float32)
    ones128 = jnp.ones((CHUNK, HALF), jnp.float32)
    b1p = jnp.broadcast_to(b1[None, :], (8, H))

    degp = _sc_degree(dst, zeros128, ones128)
    mf, crow = _tc_matmul(emb, W1, W2, b1p)
    t, dinvs = _tc_scale(mf, degp)
    s1 = _sc_propagate(t, src, dst, zeros128)
    t2 = _tc_combine(s1, t, dinvs, crow)
    s2 = _sc_propagate(t2, src, dst, zeros128)
    return _tc_finalize_bn(s2, t2, dinvs, gamma[None, :], beta[None, :])
